# Initial kernel scaffold; baseline (speedup 1.0000x reference)
import jax
import jax.numpy as jnp
from jax.experimental import pallas as pl


def kernel(x_path, x_omic1, x_omic2, x_omic3, x_omic4, x_omic5, x_omic6, params):
    raise NotImplementedError("write your pallas kernel here")


# speedup vs baseline: 6.1407x; 6.1407x over previous
import jax
import jax.numpy as jnp
from jax.experimental import pallas as pl
from jax.experimental.pallas import tpu as pltpu

DIM = 512
NP = 4096
BM = 1024
NEG = -1e30
OMIC_SIZES = (100, 200, 300, 400, 500, 600)
OMIC_PAD = 640
F32 = jnp.float32


def _elu(x):
    return jnp.where(x > 0, x, jnp.exp(jnp.minimum(x, 0.0)) - 1.0)


def _rmsnorm(x, g):
    return x * g / jnp.sqrt(jnp.mean(x * x, axis=-1, keepdims=True) + 1e-8)


def _dot(a, b):
    return jnp.dot(a, b, preferred_element_type=F32)


def _dot_t(a, b):
    return jax.lax.dot_general(a, b, (((1,), (1,)), ((), ())),
                               preferred_element_type=F32)


def _dot_c0(a, b):
    return jax.lax.dot_general(a, b, (((0,), (0,)), ((), ())),
                               preferred_element_type=F32)


def _path_enc_body(x_ref, w_ref, b_ref, h_ref, s_ref):
    i = pl.program_id(0)
    h = jnp.maximum(_dot(x_ref[...], w_ref[...]) + b_ref[...], 0.0)
    h_ref[...] = h

    @pl.when(i == 0)
    def _():
        s_ref[...] = jnp.zeros_like(s_ref)

    s_ref[...] += jnp.sum(h, axis=0, keepdims=True)


def _path_enc(x, w, b):
    return pl.pallas_call(
        _path_enc_body,
        grid=(NP // BM,),
        in_specs=[
            pl.BlockSpec((BM, 1024), lambda i: (i, 0)),
            pl.BlockSpec((1024, DIM), lambda i: (0, 0)),
            pl.BlockSpec((1, DIM), lambda i: (0, 0)),
        ],
        out_specs=[
            pl.BlockSpec((BM, DIM), lambda i: (i, 0)),
            pl.BlockSpec((1, DIM), lambda i: (0, 0)),
        ],
        out_shape=[
            jax.ShapeDtypeStruct((NP, DIM), F32),
            jax.ShapeDtypeStruct((1, DIM), F32),
        ],
    )(x, w, b.reshape(1, DIM))


def _omic_enc_body(x_ref, w1_ref, b1_ref, w2_ref, b2_ref, o_ref, s_ref):
    outs = []
    for i in range(6):
        xi = x_ref[i:i + 1, :]
        h = _elu(_dot(xi, w1_ref[i]) + b1_ref[i:i + 1, :])
        outs.append(_elu(_dot(h, w2_ref[i]) + b2_ref[i:i + 1, :]))
    o = jnp.concatenate(outs, axis=0)
    o_ref[...] = o
    s_ref[...] = jnp.sum(o, axis=0, keepdims=True)


def _omic_enc(x6, w1, b1, w2, b2):
    return pl.pallas_call(
        _omic_enc_body,
        out_shape=[
            jax.ShapeDtypeStruct((6, DIM), F32),
            jax.ShapeDtypeStruct((1, DIM), F32),
        ],
    )(x6, w1, b1, w2, b2)


def _gate(sum1, n1, sum2, n2, sim, gates):
    f = 0.5 * (sum1 / n1 + sum2 / n2)
    fn = f / (jnp.sqrt(jnp.sum(f * f)) + 1e-8)
    sn = sim / (jnp.sqrt(jnp.sum(sim * sim, axis=-1, keepdims=True)) + 1e-8)
    scores = _dot_t(fn, sn) + gates
    iota = jax.lax.broadcasted_iota(jnp.int32, (1, 4), 1)
    v1 = jnp.max(scores)
    i1 = jnp.min(jnp.where(scores == v1, iota, 9999))
    masked = jnp.where(iota == i1, NEG, scores)
    v2 = jnp.max(masked)
    i2 = jnp.min(jnp.where(masked == v2, iota, 9999))
    e2 = jnp.exp(v2 - v1)
    w1 = 1.0 / (1.0 + e2)
    w2 = e2 / (1.0 + e2)
    l = jnp.where(iota == i1, w1, 0.0) + jnp.where(iota == i2, w2, 0.0)
    ns = jnp.sum((l > 0).astype(F32))
    return jnp.concatenate(
        [l, jnp.full((1, 1), ns, F32), jnp.zeros((1, 3), F32)], axis=1)


def _a_pre_body(o_ref, sp_ref, so_ref, sim_ref, gates_ref, wk_ref, wv_ref,
                n2g_ref, s2w_ref, s2b_ref, v_ref, u_ref, wd_ref,
                k8_ref, v8_ref, bvec_ref, ctx_ref, g_ref):
    o = o_ref[...]
    z2 = jnp.zeros((2, DIM), F32)
    k8_ref[...] = jnp.concatenate([_dot(o, wk_ref[...]), z2], axis=0)
    v8_ref[...] = jnp.concatenate([_dot(o, wv_ref[...]), z2], axis=0)
    h2 = _elu(_dot(_rmsnorm(o, n2g_ref[...]), s2w_ref[...]) + s2b_ref[...])
    bvec_ref[...] = jnp.mean(h2, axis=0, keepdims=True)
    a = jnp.tanh(_dot(o, v_ref[...])) * jax.nn.sigmoid(_dot(o, u_ref[...]))
    s = jnp.sum(a * wd_ref[...], axis=1, keepdims=True)
    p = jnp.exp(s - jnp.max(s))
    attn = p / jnp.sum(p)
    ctx_ref[...] = jnp.sum(attn * o, axis=0, keepdims=True)
    g_ref[...] = _gate(sp_ref[...], 4096.0, so_ref[...], 6.0,
                       sim_ref[...], gates_ref[...])


def _a_pre(o, sum_p, sum_o, mp):
    return pl.pallas_call(
        _a_pre_body,
        out_shape=[
            jax.ShapeDtypeStruct((8, DIM), F32),
            jax.ShapeDtypeStruct((8, DIM), F32),
            jax.ShapeDtypeStruct((1, DIM), F32),
            jax.ShapeDtypeStruct((1, DIM), F32),
            jax.ShapeDtypeStruct((1, 8), F32),
        ],
    )(o, sum_p, sum_o, mp['gate']['sim'], mp['gate']['gates'].reshape(1, 4),
      mp['coa']['Wk'], mp['coa']['Wv'],
      mp['snnf']['n2_g'].reshape(1, DIM), mp['snnf']['s2_W'],
      mp['snnf']['s2_b'].reshape(1, DIM),
      mp['damisl']['V'], mp['damisl']['U'], mp['damisl']['w'].reshape(1, 256))


def _a_main_body(g_ref, p_ref, wq_ref, wo_ref, s1w_ref, s1b_ref, n1g_ref,
                 k8_ref, v8_ref, bvec_ref, ctx_ref, out_ref, s_ref):
    i = pl.program_id(0)
    x = p_ref[...]
    out_ref[...] = jnp.zeros_like(out_ref)
    l0, l1, l2, l3, ns = g_ref[0], g_ref[1], g_ref[2], g_ref[3], g_ref[4]

    @pl.when(l0 > 0)
    def _():
        q = _dot(x, wq_ref[...])
        s = _dot_t(q, k8_ref[...]) / jnp.sqrt(jnp.float32(DIM))
        col = jax.lax.broadcasted_iota(jnp.int32, s.shape, 1)
        s = jnp.where(col < 6, s, NEG)
        e = jnp.exp(s - jnp.max(s, axis=1, keepdims=True))
        attn = e / jnp.sum(e, axis=1, keepdims=True)
        y = _dot(attn, v8_ref[...])
        out_ref[...] += l0 * (x + _dot(y, wo_ref[...]))

    @pl.when(l1 > 0)
    def _():
        a = _elu(_dot(_rmsnorm(x, n1g_ref[...]), s1w_ref[...]) + s1b_ref[...])
        out_ref[...] += l1 * (a + bvec_ref[...])

    @pl.when(l2 > 0)
    def _():
        out_ref[...] += l2 * (x + ctx_ref[...])

    @pl.when(l3 > 0)
    def _():
        out_ref[...] += l3 * x

    out = out_ref[...] / ns
    out_ref[...] = out

    @pl.when(i == 0)
    def _():
        s_ref[...] = jnp.zeros_like(s_ref)

    s_ref[...] += jnp.sum(out, axis=0, keepdims=True)


def _a_main(gate, p, mp, k8, v8, bvec, ctx):
    full = lambda shape: pl.BlockSpec(shape, lambda i: (0, 0))
    return pl.pallas_call(
        _a_main_body,
        grid=(NP // BM,),
        in_specs=[
            pl.BlockSpec(memory_space=pltpu.SMEM),
            pl.BlockSpec((BM, DIM), lambda i: (i, 0)),
            full((DIM, DIM)), full((DIM, DIM)), full((DIM, DIM)),
            full((1, DIM)), full((1, DIM)),
            full((8, DIM)), full((8, DIM)), full((1, DIM)), full((1, DIM)),
        ],
        out_specs=[
            pl.BlockSpec((BM, DIM), lambda i: (i, 0)),
            pl.BlockSpec((1, DIM), lambda i: (0, 0)),
        ],
        out_shape=[
            jax.ShapeDtypeStruct((NP, DIM), F32),
            jax.ShapeDtypeStruct((1, DIM), F32),
        ],
    )(gate.reshape(8), p, mp['coa']['Wq'], mp['coa']['Wo'],
      mp['snnf']['s1_W'], mp['snnf']['s1_b'].reshape(1, DIM),
      mp['snnf']['n1_g'].reshape(1, DIM), k8, v8, bvec, ctx)


def _b_pre_body(o_ref, so_ref, sp_ref, sim_ref, gates_ref, wq_ref,
                n1g_ref, s1w_ref, s1b_ref,
                q8_ref, a6_ref, g_ref):
    o = o_ref[...]
    q8_ref[...] = jnp.concatenate(
        [_dot(o, wq_ref[...]), jnp.zeros((2, DIM), F32)], axis=0)
    a6_ref[...] = _elu(_dot(_rmsnorm(o, n1g_ref[...]), s1w_ref[...])
                       + s1b_ref[...])
    g_ref[...] = _gate(so_ref[...], 6.0, sp_ref[...], 4096.0,
                       sim_ref[...], gates_ref[...])


def _b_pre(o, sum_o, sum_p, mp):
    return pl.pallas_call(
        _b_pre_body,
        out_shape=[
            jax.ShapeDtypeStruct((8, DIM), F32),
            jax.ShapeDtypeStruct((6, DIM), F32),
            jax.ShapeDtypeStruct((1, 8), F32),
        ],
    )(o, sum_o, sum_p, mp['gate']['sim'], mp['gate']['gates'].reshape(1, 4),
      mp['coa']['Wq'], mp['snnf']['n1_g'].reshape(1, DIM),
      mp['snnf']['s1_W'], mp['snnf']['s1_b'].reshape(1, DIM))


def _b_main_body(g_ref, p_ref, o_ref, q8_ref, a6_ref,
                 wk_ref, wv_ref, wo_ref, n2g_ref, s2w_ref, s2b_ref,
                 v_ref, u_ref, wd_ref,
                 onew_ref, snew_ref,
                 accv_ref, mv_ref, dv_ref, accd_ref, md_ref, dd_ref,
                 ssum_ref):
    i = pl.program_id(0)
    n = pl.num_programs(0)
    l0, l1, l2, l3, ns = g_ref[0], g_ref[1], g_ref[2], g_ref[3], g_ref[4]

    @pl.when(i == 0)
    def _():
        accv_ref[...] = jnp.zeros_like(accv_ref)
        mv_ref[...] = jnp.full_like(mv_ref, NEG)
        dv_ref[...] = jnp.zeros_like(dv_ref)
        accd_ref[...] = jnp.zeros_like(accd_ref)
        md_ref[...] = jnp.full_like(md_ref, NEG)
        dd_ref[...] = jnp.zeros_like(dd_ref)
        ssum_ref[...] = jnp.zeros_like(ssum_ref)

    x = p_ref[...]

    @pl.when(l0 > 0)
    def _():
        k = _dot(x, wk_ref[...])
        v = _dot(x, wv_ref[...])
        s = _dot_t(q8_ref[...], k) / jnp.sqrt(jnp.float32(DIM))
        m_old = mv_ref[...]
        m_new = jnp.maximum(m_old, jnp.max(s, axis=1, keepdims=True))
        alpha = jnp.exp(m_old - m_new)
        p = jnp.exp(s - m_new)
        mv_ref[...] = m_new
        dv_ref[...] = dv_ref[...] * alpha + jnp.sum(p, axis=1, keepdims=True)
        accv_ref[...] = accv_ref[...] * alpha + _dot(p, v)

    @pl.when(l1 > 0)
    def _():
        h = _elu(_dot(_rmsnorm(x, n2g_ref[...]), s2w_ref[...]) + s2b_ref[...])
        ssum_ref[...] += jnp.sum(h, axis=0, keepdims=True)

    @pl.when(l2 > 0)
    def _():
        a = jnp.tanh(_dot(x, v_ref[...])) * jax.nn.sigmoid(_dot(x, u_ref[...]))
        s = jnp.sum(a * wd_ref[...], axis=1, keepdims=True)
        m_old = md_ref[...]
        m_new = jnp.maximum(m_old, jnp.max(s))
        alpha = jnp.exp(m_old - m_new)
        p = jnp.exp(s - m_new)
        md_ref[...] = m_new
        dd_ref[...] = dd_ref[...] * alpha + jnp.sum(p)
        accd_ref[...] = accd_ref[...] * alpha + _dot_c0(p, x)

    @pl.when(i == n - 1)
    def _():
        o = o_ref[...]
        onew_ref[...] = jnp.zeros_like(onew_ref)

        @pl.when(l0 > 0)
        def _():
            y = (accv_ref[...] / dv_ref[...])[0:6, :]
            onew_ref[...] += l0 * (o + _dot(y, wo_ref[...]))

        @pl.when(l1 > 0)
        def _():
            onew_ref[...] += l1 * (a6_ref[...] + ssum_ref[...] / 4096.0)

        @pl.when(l2 > 0)
        def _():
            onew_ref[...] += l2 * (o + accd_ref[...] / dd_ref[...])

        @pl.when(l3 > 0)
        def _():
            onew_ref[...] += l3 * o

        onew_ref[...] = onew_ref[...] / ns
        snew_ref[...] = jnp.sum(onew_ref[...], axis=0, keepdims=True)


def _b_main(gate, p, o, q8, a6, mp):
    full = lambda shape: pl.BlockSpec(shape, lambda i: (0, 0))
    return pl.pallas_call(
        _b_main_body,
        grid=(NP // BM,),
        in_specs=[
            pl.BlockSpec(memory_space=pltpu.SMEM),
            pl.BlockSpec((BM, DIM), lambda i: (i, 0)),
            full((6, DIM)), full((8, DIM)), full((6, DIM)),
            full((DIM, DIM)), full((DIM, DIM)), full((DIM, DIM)),
            full((1, DIM)), full((DIM, DIM)), full((1, DIM)),
            full((DIM, 256)), full((DIM, 256)), full((1, 256)),
        ],
        out_specs=[
            pl.BlockSpec((6, DIM), lambda i: (0, 0)),
            pl.BlockSpec((1, DIM), lambda i: (0, 0)),
        ],
        out_shape=[
            jax.ShapeDtypeStruct((6, DIM), F32),
            jax.ShapeDtypeStruct((1, DIM), F32),
        ],
        scratch_shapes=[
            pltpu.VMEM((8, DIM), F32),
            pltpu.VMEM((8, 1), F32),
            pltpu.VMEM((8, 1), F32),
            pltpu.VMEM((1, DIM), F32),
            pltpu.VMEM((1, 1), F32),
            pltpu.VMEM((1, 1), F32),
            pltpu.VMEM((1, DIM), F32),
        ],
    )(gate.reshape(8), p, o, q8, a6,
      mp['coa']['Wk'], mp['coa']['Wv'], mp['coa']['Wo'],
      mp['snnf']['n2_g'].reshape(1, DIM), mp['snnf']['s2_W'],
      mp['snnf']['s2_b'].reshape(1, DIM),
      mp['damisl']['V'], mp['damisl']['U'], mp['damisl']['w'].reshape(1, 256))


def _ln(x, g, b):
    mu = jnp.mean(x, axis=-1, keepdims=True)
    xc = x - mu
    var = jnp.mean(xc * xc, axis=-1, keepdims=True)
    return xc / jnp.sqrt(var + 1e-5) * g + b


def _attn_body(p_ref, tail_ref, lng_ref, lnb_ref, wq_ref, wk_ref, wv_ref,
               bq_ref, bk_ref, bv_ref, m8_ref, wo_ref, bo_ref,
               cw_ref, cb_ref, out_ref,
               q_ref, m_ref, d_ref, acc_ref):
    i = pl.program_id(0)
    n = pl.num_programs(0)
    g = lng_ref[...]
    b = lnb_ref[...]
    hd_scale = jnp.sqrt(jnp.float32(DIM // 8))

    @pl.when(i == 0)
    def _():
        ycls = _ln(tail_ref[0:1, :], g, b)
        q_ref[...] = _dot(ycls, wq_ref[...]) + bq_ref[...]
        m_ref[...] = jnp.full_like(m_ref, NEG)
        d_ref[...] = jnp.zeros_like(d_ref)
        acc_ref[...] = jnp.zeros_like(acc_ref)

    def upd(rows, nvalid):
        y = _ln(rows, g, b)
        k = _dot(y, wk_ref[...]) + bk_ref[...]
        v = _dot(y, wv_ref[...]) + bv_ref[...]
        s = _dot_t(k * q_ref[...], m8_ref[...]) / hd_scale
        if nvalid is not None:
            row = jax.lax.broadcasted_iota(jnp.int32, s.shape, 0)
            s = jnp.where(row < nvalid, s, NEG)
        m_old = m_ref[...]
        m_new = jnp.maximum(m_old, jnp.max(s, axis=0, keepdims=True))
        alpha = jnp.exp(m_old - m_new)
        p = jnp.exp(s - m_new)
        m_ref[...] = m_new
        d_ref[...] = d_ref[...] * alpha + jnp.sum(p, axis=0, keepdims=True)
        pb = _dot(p, m8_ref[...])
        acc_ref[...] = (acc_ref[...] * _dot(alpha, m8_ref[...])
                        + jnp.sum(pb * v, axis=0, keepdims=True))

    upd(p_ref[...], None)

    @pl.when(i == n - 1)
    def _():
        upd(tail_ref[...], 7)
        o = acc_ref[...] / _dot(d_ref[...], m8_ref[...])
        hcls = tail_ref[0:1, :] + _dot(o, wo_ref[...]) + bo_ref[...]
        out_ref[...] = _dot(hcls, cw_ref[...]) + cb_ref[...]


def _attn_cls(p, tail, sa, clf_w, clf_b):
    wqkv = sa['Wqkv']
    bqkv = sa['bqkv']
    full = lambda shape: pl.BlockSpec(shape, lambda i: (0, 0))
    return pl.pallas_call(
        _attn_body,
        grid=(NP // BM,),
        in_specs=[
            pl.BlockSpec((BM, DIM), lambda i: (i, 0)),
            full((8, DIM)), full((1, DIM)), full((1, DIM)),
            full((DIM, DIM)), full((DIM, DIM)), full((DIM, DIM)),
            full((1, DIM)), full((1, DIM)), full((1, DIM)),
            full((8, DIM)), full((DIM, DIM)), full((1, DIM)),
            full((DIM, 4)), full((1, 4)),
        ],
        out_specs=pl.BlockSpec((1, 4), lambda i: (0, 0)),
        out_shape=jax.ShapeDtypeStruct((1, 4), F32),
        scratch_shapes=[
            pltpu.VMEM((1, DIM), F32),
            pltpu.VMEM((1, 8), F32),
            pltpu.VMEM((1, 8), F32),
            pltpu.VMEM((1, DIM), F32),
        ],
    )(p, tail, sa['ln_g'].reshape(1, DIM), sa['ln_b'].reshape(1, DIM),
      wqkv[:, 0:DIM], wqkv[:, DIM:2 * DIM], wqkv[:, 2 * DIM:3 * DIM],
      bqkv[0:DIM].reshape(1, DIM), bqkv[DIM:2 * DIM].reshape(1, DIM),
      bqkv[2 * DIM:].reshape(1, DIM),
      _head_mask(), sa['Wo'], sa['bo'].reshape(1, DIM), clf_w,
      clf_b.reshape(1, 4))


def _head_mask():
    d = jnp.arange(DIM) // (DIM // 8)
    return (d[None, :] == jnp.arange(8)[:, None]).astype(F32)


def kernel(x_path, x_omic1, x_omic2, x_omic3, x_omic4, x_omic5, x_omic6,
           params):
    p = params
    h_path, sum_p = _path_enc(x_path, p['wsi_W'], p['wsi_b'])

    xo = [x_omic1, x_omic2, x_omic3, x_omic4, x_omic5, x_omic6]
    x6 = jnp.stack([jnp.pad(x, (0, OMIC_PAD - d))
                    for x, d in zip(xo, OMIC_SIZES)])
    w1 = jnp.stack([jnp.pad(s['W1'], ((0, OMIC_PAD - d), (0, 0)))
                    for s, d in zip(p['sig'], OMIC_SIZES)])
    b1 = jnp.stack([s['b1'] for s in p['sig']])
    w2 = jnp.stack([s['W2'] for s in p['sig']])
    b2 = jnp.stack([s['b2'] for s in p['sig']])
    o0, sum_o0 = _omic_enc(x6, w1, b1, w2, b2)

    k8, v8, bvec, ctx, g0 = _a_pre(o0, sum_p, sum_o0, p['mome'][0])
    p1, sum_p1 = _a_main(g0, h_path, p['mome'][0], k8, v8, bvec, ctx)

    q8, a6, g1 = _b_pre(o0, sum_o0, sum_p1, p['mome'][1])
    o1, sum_o1 = _b_main(g1, p1, o0, q8, a6, p['mome'][1])

    k8, v8, bvec, ctx, g2 = _a_pre(o1, sum_p1, sum_o1, p['mome'][2])
    p2, sum_p2 = _a_main(g2, p1, p['mome'][2], k8, v8, bvec, ctx)

    q8, a6, g3 = _b_pre(o1, sum_o1, sum_p2, p['mome'][3])
    o2, _ = _b_main(g3, p2, o1, q8, a6, p['mome'][3])

    tail = jnp.concatenate(
        [p['sa']['cls_token'][0], o2, jnp.zeros((1, DIM), F32)], axis=0)
    return _attn_cls(p2, tail, p['sa'], p['clf_W'], p['clf_b'])


# speedup vs baseline: 7.6314x; 1.2428x over previous
import jax
import jax.numpy as jnp
from jax.experimental import pallas as pl
from jax.experimental.pallas import tpu as pltpu

DIM = 512
NP = 4096
BM = 1024
BMX = 512
S = NP // BM
S0 = NP // BMX
NEG = -1e30
F32 = jnp.float32

P0 = 0
A0PRE = S0
A0 = S0 + 1
B1PRE = S0 + S + 1
B1 = S0 + S + 2
B1C = S0 + 2 * S + 2
A2PRE = S0 + 2 * S + 3
A2 = S0 + 2 * S + 4
B3PRE = S0 + 3 * S + 4
B3 = S0 + 3 * S + 5
B3C = S0 + 4 * S + 5
AT = S0 + 4 * S + 6
ATF = S0 + 5 * S + 6
NSTEPS = S0 + 5 * S + 7


def _elu(x):
    return jnp.where(x > 0, x, jnp.exp(jnp.minimum(x, 0.0)) - 1.0)


def _rmsnorm(x, g):
    return x * g / jnp.sqrt(jnp.mean(x * x, axis=-1, keepdims=True) + 1e-8)


def _dot(a, b):
    return jnp.dot(a, b, preferred_element_type=F32)


def _dot_t(a, b):
    return jax.lax.dot_general(a, b, (((1,), (1,)), ((), ())),
                               preferred_element_type=F32)


def _dot_c0(a, b):
    return jax.lax.dot_general(a, b, (((0,), (0,)), ((), ())),
                               preferred_element_type=F32)


def _ln(x, g, b):
    mu = jnp.mean(x, axis=-1, keepdims=True)
    xc = x - mu
    var = jnp.mean(xc * xc, axis=-1, keepdims=True)
    return xc / jnp.sqrt(var + 1e-5) * g + b


def _gate_vec(sum1, n1, sum2, n2, sim, gates):
    f = 0.5 * (sum1 / n1 + sum2 / n2)
    fn = f / (jnp.sqrt(jnp.sum(f * f)) + 1e-8)
    sn = sim / (jnp.sqrt(jnp.sum(sim * sim, axis=-1, keepdims=True)) + 1e-8)
    scores = _dot_t(fn, sn) + gates
    iota = jax.lax.broadcasted_iota(jnp.int32, (1, 4), 1)
    v1 = jnp.max(scores)
    i1 = jnp.min(jnp.where(scores == v1, iota, 9999))
    masked = jnp.where(iota == i1, NEG, scores)
    v2 = jnp.max(masked)
    i2 = jnp.min(jnp.where(masked == v2, iota, 9999))
    e2 = jnp.exp(v2 - v1)
    w1 = 1.0 / (1.0 + e2)
    w2 = e2 / (1.0 + e2)
    l = jnp.where(iota == i1, w1, 0.0) + jnp.where(iota == i2, w2, 0.0)
    ns = jnp.sum((l > 0).astype(F32))
    return jnp.concatenate(
        [l, jnp.full((1, 1), ns, F32), jnp.zeros((1, 3), F32)], axis=1)


_LANE8 = lambda: jax.lax.broadcasted_iota(jnp.int32, (1, 8), 1)


def _gl(g_ref, idx):
    return jnp.sum(jnp.where(_LANE8() == idx, g_ref[...], 0.0))


def _head_mask():
    d = jax.lax.broadcasted_iota(jnp.int32, (8, DIM), 1) // (DIM // 8)
    h = jax.lax.broadcasted_iota(jnp.int32, (8, DIM), 0)
    return (d == h).astype(F32)


def _fwd_body(*refs):
    (x_ref, wsiw_ref, wsib_ref) = refs[0:3]
    xo = refs[3:9]
    w1 = refs[9:15]
    b1 = refs[15:21]
    w2 = refs[21:27]
    b2 = refs[27:33]
    mome = [refs[33 + 15 * j: 33 + 15 * (j + 1)] for j in range(4)]
    (cls_ref, lng_ref, lnb_ref, wqkv_ref, bqkv_ref, sawo_ref, sabo_ref,
     clfw_ref, clfb_ref) = refs[93:102]
    out_ref = refs[102]
    (p_ref, o_ref, sp_ref, so_ref, k8_ref, v8_ref, bvec_ref, ctx_ref,
     q8_ref, a6_ref, g_ref, accv_ref, mv_ref, dv_ref, accd_ref, md_ref,
     dd_ref, ssum_ref, qv_ref, am_ref, ad_ref, aacc_ref, tail_ref) = \
        refs[103:126]

    s = pl.program_id(0)
    row8 = jax.lax.broadcasted_iota(jnp.int32, (8, 1), 0)
    mask6 = row8 < 6
    sqd = jnp.sqrt(jnp.float32(DIM))

    @pl.when(s < S0)
    def _():
        h = jnp.maximum(_dot(x_ref[...], wsiw_ref[...]) + wsib_ref[...], 0.0)
        p_ref[pl.ds(s * BMX, BMX), :] = h

        @pl.when(s == 0)
        def _():
            sp_ref[...] = jnp.zeros_like(sp_ref)
            rows = []
            for i in range(6):
                hh = _elu(_dot(xo[i][...], w1[i][...]) + b1[i][...])
                rows.append(_elu(_dot(hh, w2[i][...]) + b2[i][...]))
            o = jnp.concatenate(rows + [jnp.zeros((2, DIM), F32)], axis=0)
            o_ref[...] = o
            so_ref[...] = jnp.sum(o, axis=0, keepdims=True)

        sp_ref[...] += jnp.sum(h, axis=0, keepdims=True)

    def a_pre(mp):
        (sim, gates, wq, wk, wv, wo, n1g, s1w, s1b, n2g, s2w, s2b,
         vv, uu, wd) = mp
        o = o_ref[...]
        k8_ref[...] = _dot(o, wk[...])
        v8_ref[...] = _dot(o, wv[...])
        h2 = _elu(_dot(_rmsnorm(o, n2g[...]), s2w[...]) + s2b[...])
        h2 = jnp.where(mask6, h2, 0.0)
        bvec_ref[...] = jnp.sum(h2, axis=0, keepdims=True) / 6.0
        a = jnp.tanh(_dot(o, vv[...])) * jax.nn.sigmoid(_dot(o, uu[...]))
        sd = jnp.sum(a * wd[...], axis=1, keepdims=True)
        sd = jnp.where(mask6, sd, NEG)
        pd = jnp.exp(sd - jnp.max(sd))
        attn = pd / jnp.sum(pd)
        ctx_ref[...] = jnp.sum(attn * o, axis=0, keepdims=True)
        g_ref[...] = _gate_vec(sp_ref[...], 4096.0, so_ref[...], 6.0,
                               sim[...], gates[...])

    def a_main(mp, base):
        (sim, gates, wq, wk, wv, wo, n1g, s1w, s1b, n2g, s2w, s2b,
         vv, uu, wd) = mp
        blk = (s - base) * BM
        x = p_ref[pl.ds(blk, BM), :]
        l0, l1, l2, l3 = (_gl(g_ref, 0), _gl(g_ref, 1),
                          _gl(g_ref, 2), _gl(g_ref, 3))
        ns = _gl(g_ref, 4)
        p_ref[pl.ds(blk, BM), :] = jnp.zeros((BM, DIM), F32)

        @pl.when(l0 > 0)
        def _():
            q = _dot(x, wq[...])
            sc = _dot_t(q, k8_ref[...]) / sqd
            col = jax.lax.broadcasted_iota(jnp.int32, sc.shape, 1)
            sc = jnp.where(col < 6, sc, NEG)
            e = jnp.exp(sc - jnp.max(sc, axis=1, keepdims=True))
            attn = e / jnp.sum(e, axis=1, keepdims=True)
            y = _dot(attn, v8_ref[...])
            p_ref[pl.ds(blk, BM), :] += l0 * (x + _dot(y, wo[...]))

        @pl.when(l1 > 0)
        def _():
            a = _elu(_dot(_rmsnorm(x, n1g[...]), s1w[...]) + s1b[...])
            p_ref[pl.ds(blk, BM), :] += l1 * (a + bvec_ref[...])

        @pl.when(l2 > 0)
        def _():
            p_ref[pl.ds(blk, BM), :] += l2 * (x + ctx_ref[...])

        @pl.when(l3 > 0)
        def _():
            p_ref[pl.ds(blk, BM), :] += l3 * x

        newx = p_ref[pl.ds(blk, BM), :] / ns
        p_ref[pl.ds(blk, BM), :] = newx

        @pl.when(s == base)
        def _():
            sp_ref[...] = jnp.zeros_like(sp_ref)

        sp_ref[...] += jnp.sum(newx, axis=0, keepdims=True)

    def b_pre(mp):
        (sim, gates, wq, wk, wv, wo, n1g, s1w, s1b, n2g, s2w, s2b,
         vv, uu, wd) = mp
        o = o_ref[...]
        q8_ref[...] = _dot(o, wq[...])
        a6 = _elu(_dot(_rmsnorm(o, n1g[...]), s1w[...]) + s1b[...])
        a6_ref[...] = jnp.where(mask6, a6, 0.0)
        g_ref[...] = _gate_vec(so_ref[...], 6.0, sp_ref[...], 4096.0,
                               sim[...], gates[...])
        accv_ref[...] = jnp.zeros_like(accv_ref)
        mv_ref[...] = jnp.full_like(mv_ref, NEG)
        dv_ref[...] = jnp.zeros_like(dv_ref)
        accd_ref[...] = jnp.zeros_like(accd_ref)
        md_ref[...] = jnp.full_like(md_ref, NEG)
        dd_ref[...] = jnp.zeros_like(dd_ref)
        ssum_ref[...] = jnp.zeros_like(ssum_ref)

    def b_acc(mp, base):
        (sim, gates, wq, wk, wv, wo, n1g, s1w, s1b, n2g, s2w, s2b,
         vv, uu, wd) = mp
        blk = (s - base) * BM
        x = p_ref[pl.ds(blk, BM), :]
        l0, l1, l2 = _gl(g_ref, 0), _gl(g_ref, 1), _gl(g_ref, 2)

        @pl.when(l0 > 0)
        def _():
            k = _dot(x, wk[...])
            v = _dot(x, wv[...])
            sc = _dot_t(q8_ref[...], k) / sqd
            m_old = mv_ref[...]
            m_new = jnp.maximum(m_old, jnp.max(sc, axis=1, keepdims=True))
            alpha = jnp.exp(m_old - m_new)
            pp = jnp.exp(sc - m_new)
            mv_ref[...] = m_new
            dv_ref[...] = dv_ref[...] * alpha + jnp.sum(pp, axis=1,
                                                        keepdims=True)
            accv_ref[...] = accv_ref[...] * alpha + _dot(pp, v)

        @pl.when(l1 > 0)
        def _():
            h = _elu(_dot(_rmsnorm(x, n2g[...]), s2w[...]) + s2b[...])
            ssum_ref[...] += jnp.sum(h, axis=0, keepdims=True)

        @pl.when(l2 > 0)
        def _():
            a = jnp.tanh(_dot(x, vv[...])) * jax.nn.sigmoid(_dot(x, uu[...]))
            sc = jnp.sum(a * wd[...], axis=1, keepdims=True)
            m_old = md_ref[...]
            m_new = jnp.maximum(m_old, jnp.max(sc))
            alpha = jnp.exp(m_old - m_new)
            pp = jnp.exp(sc - m_new)
            md_ref[...] = m_new
            dd_ref[...] = dd_ref[...] * alpha + jnp.sum(pp)
            accd_ref[...] = accd_ref[...] * alpha + _dot_c0(pp, x)

    def b_comb(mp):
        (sim, gates, wq, wk, wv, wo, n1g, s1w, s1b, n2g, s2w, s2b,
         vv, uu, wd) = mp
        o = o_ref[...]
        l0, l1, l2, l3 = (_gl(g_ref, 0), _gl(g_ref, 1),
                          _gl(g_ref, 2), _gl(g_ref, 3))
        ns = _gl(g_ref, 4)
        o_ref[...] = jnp.zeros_like(o_ref)

        @pl.when(l0 > 0)
        def _():
            y = accv_ref[...] / dv_ref[...]
            o_ref[...] += l0 * (o + _dot(y, wo[...]))

        @pl.when(l1 > 0)
        def _():
            o_ref[...] += l1 * (a6_ref[...] + ssum_ref[...] / 4096.0)

        @pl.when(l2 > 0)
        def _():
            o_ref[...] += l2 * (o + accd_ref[...] / dd_ref[...])

        @pl.when(l3 > 0)
        def _():
            o_ref[...] += l3 * o

        onew = jnp.where(mask6, o_ref[...] / ns, 0.0)
        o_ref[...] = onew
        so_ref[...] = jnp.sum(onew, axis=0, keepdims=True)

    m0, m1, m2, m3 = mome

    @pl.when(s == A0PRE)
    def _():
        a_pre(m0)

    @pl.when((s >= A0) & (s < A0 + S))
    def _():
        a_main(m0, A0)

    @pl.when(s == B1PRE)
    def _():
        b_pre(m1)

    @pl.when((s >= B1) & (s < B1 + S))
    def _():
        b_acc(m1, B1)

    @pl.when(s == B1C)
    def _():
        b_comb(m1)

    @pl.when(s == A2PRE)
    def _():
        a_pre(m2)

    @pl.when((s >= A2) & (s < A2 + S))
    def _():
        a_main(m2, A2)

    @pl.when(s == B3PRE)
    def _():
        b_pre(m3)

    @pl.when((s >= B3) & (s < B3 + S))
    def _():
        b_acc(m3, B3)

    hd_scale = jnp.sqrt(jnp.float32(DIM // 8))

    @pl.when(s == B3C)
    def _():
        b_comb(m3)
        tail_ref[0:1, :] = cls_ref[...]
        tail_ref[1:7, :] = o_ref[0:6, :]
        tail_ref[7:8, :] = jnp.zeros((1, DIM), F32)
        ycls = _ln(cls_ref[...], lng_ref[...], lnb_ref[...])
        qv_ref[...] = _dot(ycls, wqkv_ref[:, 0:DIM]) + bqkv_ref[:, 0:DIM]
        am_ref[...] = jnp.full_like(am_ref, NEG)
        ad_ref[...] = jnp.zeros_like(ad_ref)
        aacc_ref[...] = jnp.zeros_like(aacc_ref)

    def attn_upd(rows, nvalid):
        m8 = _head_mask()
        y = _ln(rows, lng_ref[...], lnb_ref[...])
        k = _dot(y, wqkv_ref[:, DIM:2 * DIM]) + bqkv_ref[:, DIM:2 * DIM]
        v = _dot(y, wqkv_ref[:, 2 * DIM:]) + bqkv_ref[:, 2 * DIM:]
        sc = _dot_t(k * qv_ref[...], m8) / hd_scale
        if nvalid is not None:
            row = jax.lax.broadcasted_iota(jnp.int32, sc.shape, 0)
            sc = jnp.where(row < nvalid, sc, NEG)
        m_old = am_ref[...]
        m_new = jnp.maximum(m_old, jnp.max(sc, axis=0, keepdims=True))
        alpha = jnp.exp(m_old - m_new)
        pp = jnp.exp(sc - m_new)
        am_ref[...] = m_new
        ad_ref[...] = ad_ref[...] * alpha + jnp.sum(pp, axis=0, keepdims=True)
        pb = _dot(pp, m8)
        aacc_ref[...] = (aacc_ref[...] * _dot(alpha, m8)
                         + jnp.sum(pb * v, axis=0, keepdims=True))

    @pl.when((s >= AT) & (s < AT + S))
    def _():
        attn_upd(p_ref[pl.ds((s - AT) * BM, BM), :], None)

    @pl.when(s == ATF)
    def _():
        attn_upd(tail_ref[...], 7)
        m8 = _head_mask()
        o = aacc_ref[...] / _dot(ad_ref[...], m8)
        hcls = tail_ref[0:1, :] + _dot(o, sawo_ref[...]) + sabo_ref[...]
        out_ref[...] = _dot(hcls, clfw_ref[...]) + clfb_ref[...]


def kernel(x_path, x_omic1, x_omic2, x_omic3, x_omic4, x_omic5, x_omic6,
           params):
    p = params
    xo = [x_omic1, x_omic2, x_omic3, x_omic4, x_omic5, x_omic6]

    args = [x_path, p['wsi_W'], p['wsi_b'].reshape(1, DIM)]
    args += [x.reshape(1, -1) for x in xo]
    args += [s['W1'] for s in p['sig']]
    args += [s['b1'].reshape(1, DIM) for s in p['sig']]
    args += [s['W2'] for s in p['sig']]
    args += [s['b2'].reshape(1, DIM) for s in p['sig']]
    for j in range(4):
        m = p['mome'][j]
        args += [m['gate']['sim'], m['gate']['gates'].reshape(1, 4),
                 m['coa']['Wq'], m['coa']['Wk'], m['coa']['Wv'],
                 m['coa']['Wo'],
                 m['snnf']['n1_g'].reshape(1, DIM), m['snnf']['s1_W'],
                 m['snnf']['s1_b'].reshape(1, DIM),
                 m['snnf']['n2_g'].reshape(1, DIM), m['snnf']['s2_W'],
                 m['snnf']['s2_b'].reshape(1, DIM),
                 m['damisl']['V'], m['damisl']['U'],
                 m['damisl']['w'].reshape(1, 256)]
    sa = p['sa']
    args += [sa['cls_token'].reshape(1, DIM),
             sa['ln_g'].reshape(1, DIM), sa['ln_b'].reshape(1, DIM),
             sa['Wqkv'], sa['bqkv'].reshape(1, 3 * DIM),
             sa['Wo'], sa['bo'].reshape(1, DIM),
             p['clf_W'], p['clf_b'].reshape(1, 4)]

    in_specs = [pl.BlockSpec((BMX, 1024),
                             lambda s: (jnp.minimum(s, S0 - 1), 0))]
    for a in args[1:]:
        in_specs.append(pl.BlockSpec(a.shape, lambda s: (0, 0)))

    scratch_shapes = [
        pltpu.VMEM((NP, DIM), F32),
        pltpu.VMEM((8, DIM), F32),
        pltpu.VMEM((1, DIM), F32),
        pltpu.VMEM((1, DIM), F32),
        pltpu.VMEM((8, DIM), F32),
        pltpu.VMEM((8, DIM), F32),
        pltpu.VMEM((1, DIM), F32),
        pltpu.VMEM((1, DIM), F32),
        pltpu.VMEM((8, DIM), F32),
        pltpu.VMEM((8, DIM), F32),
        pltpu.VMEM((1, 8), F32),
        pltpu.VMEM((8, DIM), F32),
        pltpu.VMEM((8, 1), F32),
        pltpu.VMEM((8, 1), F32),
        pltpu.VMEM((1, DIM), F32),
        pltpu.VMEM((1, 1), F32),
        pltpu.VMEM((1, 1), F32),
        pltpu.VMEM((1, DIM), F32),
        pltpu.VMEM((1, DIM), F32),
        pltpu.VMEM((1, 8), F32),
        pltpu.VMEM((1, 8), F32),
        pltpu.VMEM((1, DIM), F32),
        pltpu.VMEM((8, DIM), F32),
    ]

    return pl.pallas_call(
        _fwd_body,
        grid=(NSTEPS,),
        in_specs=in_specs,
        out_specs=pl.BlockSpec((1, 4), lambda s: (0, 0)),
        out_shape=jax.ShapeDtypeStruct((1, 4), F32),
        scratch_shapes=scratch_shapes,
        compiler_params=pltpu.CompilerParams(vmem_limit_bytes=100 * 2**20),
    )(*args)


# speedup vs baseline: 8.0686x; 1.0573x over previous
import jax
import jax.numpy as jnp
from jax.experimental import pallas as pl
from jax.experimental.pallas import tpu as pltpu

DIM = 512
NP = 4096
BM = 1024
BMX = 512
S = NP // BM
S0 = NP // BMX
NEG = -1e30
F32 = jnp.float32

A0 = S0
B1 = S0 + S
A2 = S0 + 2 * S
B3 = S0 + 3 * S
NSTEPS = S0 + 4 * S


def _elu(x):
    return jnp.where(x > 0, x, jnp.exp(jnp.minimum(x, 0.0)) - 1.0)


def _rmsnorm(x, g):
    return x * g / jnp.sqrt(jnp.mean(x * x, axis=-1, keepdims=True) + 1e-8)


def _dot(a, b):
    return jnp.dot(a, b, preferred_element_type=F32)


def _dot_t(a, b):
    return jax.lax.dot_general(a, b, (((1,), (1,)), ((), ())),
                               preferred_element_type=F32)


def _dot_c0(a, b):
    return jax.lax.dot_general(a, b, (((0,), (0,)), ((), ())),
                               preferred_element_type=F32)


def _ln(x, g, b):
    mu = jnp.mean(x, axis=-1, keepdims=True)
    xc = x - mu
    var = jnp.mean(xc * xc, axis=-1, keepdims=True)
    return xc / jnp.sqrt(var + 1e-5) * g + b


def _gate_vec(sum1, n1, sum2, n2, sim, gates):
    f = 0.5 * (sum1 / n1 + sum2 / n2)
    fn = f / (jnp.sqrt(jnp.sum(f * f)) + 1e-8)
    sn = sim / (jnp.sqrt(jnp.sum(sim * sim, axis=-1, keepdims=True)) + 1e-8)
    scores = _dot_t(fn, sn) + gates
    iota = jax.lax.broadcasted_iota(jnp.int32, (1, 4), 1)
    v1 = jnp.max(scores)
    i1 = jnp.min(jnp.where(scores == v1, iota, 9999))
    masked = jnp.where(iota == i1, NEG, scores)
    v2 = jnp.max(masked)
    i2 = jnp.min(jnp.where(masked == v2, iota, 9999))
    e2 = jnp.exp(v2 - v1)
    w1 = 1.0 / (1.0 + e2)
    w2 = e2 / (1.0 + e2)
    l = jnp.where(iota == i1, w1, 0.0) + jnp.where(iota == i2, w2, 0.0)
    ns = jnp.sum((l > 0).astype(F32))
    return jnp.concatenate(
        [l, jnp.full((1, 1), ns, F32), jnp.zeros((1, 3), F32)], axis=1)


def _gl(g_ref, idx):
    lane = jax.lax.broadcasted_iota(jnp.int32, (1, 8), 1)
    return jnp.sum(jnp.where(lane == idx, g_ref[...], 0.0))


def _head_mask():
    d = jax.lax.broadcasted_iota(jnp.int32, (8, DIM), 1) // (DIM // 8)
    h = jax.lax.broadcasted_iota(jnp.int32, (8, DIM), 0)
    return (d == h).astype(F32)


def _fwd_body(*refs):
    (x_ref, wsiw_ref, wsib_ref) = refs[0:3]
    xo = refs[3:9]
    w1 = refs[9:15]
    b1 = refs[15:21]
    w2 = refs[21:27]
    b2 = refs[27:33]
    mome = [refs[33 + 15 * j: 33 + 15 * (j + 1)] for j in range(4)]
    (cls_ref, lng_ref, lnb_ref, wqkv_ref, bqkv_ref, sawo_ref, sabo_ref,
     clfw_ref, clfb_ref) = refs[93:102]
    out_ref = refs[102]
    (p_ref, o_ref, sp_ref, so_ref, k8_ref, v8_ref, bvec_ref, ctx_ref,
     q8_ref, a6_ref, g_ref, accv_ref, mv_ref, dv_ref, accd_ref, md_ref,
     dd_ref, ssum_ref, qv_ref, am_ref, ad_ref, aacc_ref, tail_ref) = \
        refs[103:126]

    s = pl.program_id(0)
    row8 = jax.lax.broadcasted_iota(jnp.int32, (8, 1), 0)
    mask6 = row8 < 6
    sqd = jnp.sqrt(jnp.float32(DIM))

    @pl.when(s < S0)
    def _():
        h = jnp.maximum(_dot(x_ref[...], wsiw_ref[...]) + wsib_ref[...], 0.0)
        p_ref[pl.ds(s * BMX, BMX), :] = h

        @pl.when(s == 0)
        def _():
            sp_ref[...] = jnp.zeros_like(sp_ref)
            rows = []
            for i in range(6):
                hh = _elu(_dot(xo[i][...], w1[i][...]) + b1[i][...])
                rows.append(_elu(_dot(hh, w2[i][...]) + b2[i][...]))
            o = jnp.concatenate(rows + [jnp.zeros((2, DIM), F32)], axis=0)
            o_ref[...] = o
            so_ref[...] = jnp.sum(o, axis=0, keepdims=True)

        sp_ref[...] += jnp.sum(h, axis=0, keepdims=True)

    def a_pre(mp):
        (sim, gates, wq, wk, wv, wo, n1g, s1w, s1b, n2g, s2w, s2b,
         vv, uu, wd) = mp
        o = o_ref[...]
        k8_ref[...] = _dot(o, wk[...])
        v8_ref[...] = _dot(o, wv[...])
        h2 = _elu(_dot(_rmsnorm(o, n2g[...]), s2w[...]) + s2b[...])
        h2 = jnp.where(mask6, h2, 0.0)
        bvec_ref[...] = jnp.sum(h2, axis=0, keepdims=True) / 6.0
        a = jnp.tanh(_dot(o, vv[...])) * jax.nn.sigmoid(_dot(o, uu[...]))
        sd = jnp.sum(a * wd[...], axis=1, keepdims=True)
        sd = jnp.where(mask6, sd, NEG)
        pd = jnp.exp(sd - jnp.max(sd))
        attn = pd / jnp.sum(pd)
        ctx_ref[...] = jnp.sum(attn * o, axis=0, keepdims=True)
        g_ref[...] = _gate_vec(sp_ref[...], 4096.0, so_ref[...], 6.0,
                               sim[...], gates[...])

    def a_main(mp, base):
        (sim, gates, wq, wk, wv, wo, n1g, s1w, s1b, n2g, s2w, s2b,
         vv, uu, wd) = mp
        blk = (s - base) * BM
        x = p_ref[pl.ds(blk, BM), :]
        l0, l1, l2, l3 = (_gl(g_ref, 0), _gl(g_ref, 1),
                          _gl(g_ref, 2), _gl(g_ref, 3))
        ns = _gl(g_ref, 4)
        p_ref[pl.ds(blk, BM), :] = ((l0 + l2 + l3) / ns) * x \
            + (l2 / ns) * ctx_ref[...]

        @pl.when(l0 > 0)
        def _():
            q = _dot(x, wq[...])
            sc = _dot_t(q, k8_ref[...]) / sqd
            col = jax.lax.broadcasted_iota(jnp.int32, sc.shape, 1)
            sc = jnp.where(col < 6, sc, NEG)
            e = jnp.exp(sc - jnp.max(sc, axis=1, keepdims=True))
            attn = e / jnp.sum(e, axis=1, keepdims=True)
            y = _dot(attn, v8_ref[...])
            p_ref[pl.ds(blk, BM), :] += (l0 / ns) * _dot(y, wo[...])

        @pl.when(l1 > 0)
        def _():
            a = _elu(_dot(_rmsnorm(x, n1g[...]), s1w[...]) + s1b[...])
            p_ref[pl.ds(blk, BM), :] += (l1 / ns) * (a + bvec_ref[...])

        @pl.when(s == base)
        def _():
            sp_ref[...] = jnp.zeros_like(sp_ref)

        sp_ref[...] += jnp.sum(p_ref[pl.ds(blk, BM), :], axis=0,
                               keepdims=True)

    def b_pre(mp):
        (sim, gates, wq, wk, wv, wo, n1g, s1w, s1b, n2g, s2w, s2b,
         vv, uu, wd) = mp
        o = o_ref[...]
        q8_ref[...] = _dot(o, wq[...])
        a6 = _elu(_dot(_rmsnorm(o, n1g[...]), s1w[...]) + s1b[...])
        a6_ref[...] = jnp.where(mask6, a6, 0.0)
        g_ref[...] = _gate_vec(so_ref[...], 6.0, sp_ref[...], 4096.0,
                               sim[...], gates[...])
        accv_ref[...] = jnp.zeros_like(accv_ref)
        mv_ref[...] = jnp.full_like(mv_ref, NEG)
        dv_ref[...] = jnp.zeros_like(dv_ref)
        accd_ref[...] = jnp.zeros_like(accd_ref)
        md_ref[...] = jnp.full_like(md_ref, NEG)
        dd_ref[...] = jnp.zeros_like(dd_ref)
        ssum_ref[...] = jnp.zeros_like(ssum_ref)

    def b_acc(mp, x):
        (sim, gates, wq, wk, wv, wo, n1g, s1w, s1b, n2g, s2w, s2b,
         vv, uu, wd) = mp
        l0, l1, l2 = _gl(g_ref, 0), _gl(g_ref, 1), _gl(g_ref, 2)

        @pl.when(l0 > 0)
        def _():
            k = _dot(x, wk[...])
            v = _dot(x, wv[...])
            sc = _dot_t(q8_ref[...], k) / sqd
            m_old = mv_ref[...]
            m_new = jnp.maximum(m_old, jnp.max(sc, axis=1, keepdims=True))
            alpha = jnp.exp(m_old - m_new)
            pp = jnp.exp(sc - m_new)
            mv_ref[...] = m_new
            dv_ref[...] = dv_ref[...] * alpha + jnp.sum(pp, axis=1,
                                                        keepdims=True)
            accv_ref[...] = accv_ref[...] * alpha + _dot(pp, v)

        @pl.when(l1 > 0)
        def _():
            h = _elu(_dot(_rmsnorm(x, n2g[...]), s2w[...]) + s2b[...])
            ssum_ref[...] += jnp.sum(h, axis=0, keepdims=True)

        @pl.when(l2 > 0)
        def _():
            a = jnp.tanh(_dot(x, vv[...])) * jax.nn.sigmoid(_dot(x, uu[...]))
            sc = jnp.sum(a * wd[...], axis=1, keepdims=True)
            m_old = md_ref[...]
            m_new = jnp.maximum(m_old, jnp.max(sc))
            alpha = jnp.exp(m_old - m_new)
            pp = jnp.exp(sc - m_new)
            md_ref[...] = m_new
            dd_ref[...] = dd_ref[...] * alpha + jnp.sum(pp)
            accd_ref[...] = accd_ref[...] * alpha + _dot_c0(pp, x)

    def b_comb(mp):
        (sim, gates, wq, wk, wv, wo, n1g, s1w, s1b, n2g, s2w, s2b,
         vv, uu, wd) = mp
        o = o_ref[...]
        l0, l1, l2, l3 = (_gl(g_ref, 0), _gl(g_ref, 1),
                          _gl(g_ref, 2), _gl(g_ref, 3))
        ns = _gl(g_ref, 4)
        o_ref[...] = ((l0 + l2 + l3) / ns) * o \
            + (l2 / ns) * accd_ref[...] / dd_ref[...]

        @pl.when(l0 > 0)
        def _():
            y = accv_ref[...] / dv_ref[...]
            o_ref[...] += (l0 / ns) * _dot(y, wo[...])

        @pl.when(l1 > 0)
        def _():
            o_ref[...] += (l1 / ns) * (a6_ref[...] + ssum_ref[...] / 4096.0)

        onew = jnp.where(mask6, o_ref[...], 0.0)
        o_ref[...] = onew
        so_ref[...] = jnp.sum(onew, axis=0, keepdims=True)

    hd_scale = jnp.sqrt(jnp.float32(DIM // 8))

    def attn_upd(rows, nvalid):
        m8 = _head_mask()
        y = _ln(rows, lng_ref[...], lnb_ref[...])
        k = _dot(y, wqkv_ref[:, DIM:2 * DIM]) + bqkv_ref[:, DIM:2 * DIM]
        v = _dot(y, wqkv_ref[:, 2 * DIM:]) + bqkv_ref[:, 2 * DIM:]
        sc = _dot_t(k * qv_ref[...], m8) / hd_scale
        if nvalid is not None:
            row = jax.lax.broadcasted_iota(jnp.int32, sc.shape, 0)
            sc = jnp.where(row < nvalid, sc, NEG)
        m_old = am_ref[...]
        m_new = jnp.maximum(m_old, jnp.max(sc, axis=0, keepdims=True))
        alpha = jnp.exp(m_old - m_new)
        pp = jnp.exp(sc - m_new)
        am_ref[...] = m_new
        ad_ref[...] = ad_ref[...] * alpha + jnp.sum(pp, axis=0, keepdims=True)
        pb = _dot(pp, m8)
        aacc_ref[...] = (aacc_ref[...] * _dot(alpha, m8)
                         + jnp.sum(pb * v, axis=0, keepdims=True))

    m0, m1, m2, m3 = mome

    @pl.when(s == S0 - 1)
    def _():
        a_pre(m0)

    @pl.when((s >= A0) & (s < A0 + S))
    def _():
        a_main(m0, A0)

    @pl.when(s == A0 + S - 1)
    def _():
        b_pre(m1)

    @pl.when((s >= B1) & (s < B1 + S))
    def _():
        b_acc(m1, p_ref[pl.ds((s - B1) * BM, BM), :])

    @pl.when(s == B1 + S - 1)
    def _():
        b_comb(m1)
        a_pre(m2)

    @pl.when((s >= A2) & (s < A2 + S))
    def _():
        a_main(m2, A2)

    @pl.when(s == A2 + S - 1)
    def _():
        b_pre(m3)
        ycls = _ln(cls_ref[...], lng_ref[...], lnb_ref[...])
        qv_ref[...] = _dot(ycls, wqkv_ref[:, 0:DIM]) + bqkv_ref[:, 0:DIM]
        am_ref[...] = jnp.full_like(am_ref, NEG)
        ad_ref[...] = jnp.zeros_like(ad_ref)
        aacc_ref[...] = jnp.zeros_like(aacc_ref)

    @pl.when((s >= B3) & (s < B3 + S))
    def _():
        x = p_ref[pl.ds((s - B3) * BM, BM), :]
        b_acc(m3, x)
        attn_upd(x, None)

    @pl.when(s == B3 + S - 1)
    def _():
        b_comb(m3)
        tail_ref[0:1, :] = cls_ref[...]
        tail_ref[1:7, :] = o_ref[0:6, :]
        tail_ref[7:8, :] = jnp.zeros((1, DIM), F32)
        attn_upd(tail_ref[...], 7)
        m8 = _head_mask()
        o = aacc_ref[...] / _dot(ad_ref[...], m8)
        hcls = tail_ref[0:1, :] + _dot(o, sawo_ref[...]) + sabo_ref[...]
        out_ref[...] = _dot(hcls, clfw_ref[...]) + clfb_ref[...]


def kernel(x_path, x_omic1, x_omic2, x_omic3, x_omic4, x_omic5, x_omic6,
           params):
    p = params
    xo = [x_omic1, x_omic2, x_omic3, x_omic4, x_omic5, x_omic6]

    args = [x_path, p['wsi_W'], p['wsi_b'].reshape(1, DIM)]
    args += [x.reshape(1, -1) for x in xo]
    args += [s['W1'] for s in p['sig']]
    args += [s['b1'].reshape(1, DIM) for s in p['sig']]
    args += [s['W2'] for s in p['sig']]
    args += [s['b2'].reshape(1, DIM) for s in p['sig']]
    for j in range(4):
        m = p['mome'][j]
        args += [m['gate']['sim'], m['gate']['gates'].reshape(1, 4),
                 m['coa']['Wq'], m['coa']['Wk'], m['coa']['Wv'],
                 m['coa']['Wo'],
                 m['snnf']['n1_g'].reshape(1, DIM), m['snnf']['s1_W'],
                 m['snnf']['s1_b'].reshape(1, DIM),
                 m['snnf']['n2_g'].reshape(1, DIM), m['snnf']['s2_W'],
                 m['snnf']['s2_b'].reshape(1, DIM),
                 m['damisl']['V'], m['damisl']['U'],
                 m['damisl']['w'].reshape(1, 256)]
    sa = p['sa']
    args += [sa['cls_token'].reshape(1, DIM),
             sa['ln_g'].reshape(1, DIM), sa['ln_b'].reshape(1, DIM),
             sa['Wqkv'], sa['bqkv'].reshape(1, 3 * DIM),
             sa['Wo'], sa['bo'].reshape(1, DIM),
             p['clf_W'], p['clf_b'].reshape(1, 4)]

    in_specs = [pl.BlockSpec((BMX, 1024),
                             lambda s: (jnp.minimum(s, S0 - 1), 0))]
    for a in args[1:]:
        in_specs.append(pl.BlockSpec(a.shape, lambda s: (0, 0)))

    scratch_shapes = [
        pltpu.VMEM((NP, DIM), F32),
        pltpu.VMEM((8, DIM), F32),
        pltpu.VMEM((1, DIM), F32),
        pltpu.VMEM((1, DIM), F32),
        pltpu.VMEM((8, DIM), F32),
        pltpu.VMEM((8, DIM), F32),
        pltpu.VMEM((1, DIM), F32),
        pltpu.VMEM((1, DIM), F32),
        pltpu.VMEM((8, DIM), F32),
        pltpu.VMEM((8, DIM), F32),
        pltpu.VMEM((1, 8), F32),
        pltpu.VMEM((8, DIM), F32),
        pltpu.VMEM((8, 1), F32),
        pltpu.VMEM((8, 1), F32),
        pltpu.VMEM((1, DIM), F32),
        pltpu.VMEM((1, 1), F32),
        pltpu.VMEM((1, 1), F32),
        pltpu.VMEM((1, DIM), F32),
        pltpu.VMEM((1, DIM), F32),
        pltpu.VMEM((1, 8), F32),
        pltpu.VMEM((1, 8), F32),
        pltpu.VMEM((1, DIM), F32),
        pltpu.VMEM((8, DIM), F32),
    ]

    return pl.pallas_call(
        _fwd_body,
        grid=(NSTEPS,),
        in_specs=in_specs,
        out_specs=pl.BlockSpec((1, 4), lambda s: (0, 0)),
        out_shape=jax.ShapeDtypeStruct((1, 4), F32),
        scratch_shapes=scratch_shapes,
        compiler_params=pltpu.CompilerParams(vmem_limit_bytes=100 * 2**20),
    )(*args)


# speedup vs baseline: 8.2316x; 1.0202x over previous
import jax
import jax.numpy as jnp
from jax.experimental import pallas as pl
from jax.experimental.pallas import tpu as pltpu

DIM = 512
NP = 4096
BM = 1024
BMX = 512
S = NP // BM
S0 = NP // BMX
NEG = -1e30
F32 = jnp.float32

A0 = S0
B1 = S0 + S
A2 = S0 + 2 * S
B3 = S0 + 3 * S
NSTEPS = S0 + 4 * S


def _elu(x):
    return jnp.where(x > 0, x, jnp.exp(jnp.minimum(x, 0.0)) - 1.0)


def _rmsnorm(x, g):
    return x * g / jnp.sqrt(jnp.mean(x * x, axis=-1, keepdims=True) + 1e-8)


def _dot(a, b):
    return jnp.dot(a, b, preferred_element_type=F32)


def _dot_t(a, b):
    return jax.lax.dot_general(a, b, (((1,), (1,)), ((), ())),
                               preferred_element_type=F32)


def _dot_c0(a, b):
    return jax.lax.dot_general(a, b, (((0,), (0,)), ((), ())),
                               preferred_element_type=F32)


def _ln(x, g, b):
    mu = jnp.mean(x, axis=-1, keepdims=True)
    xc = x - mu
    var = jnp.mean(xc * xc, axis=-1, keepdims=True)
    return xc / jnp.sqrt(var + 1e-5) * g + b


def _gate_vec(sum1, n1, sum2, n2, sim, gates):
    f = 0.5 * (sum1 / n1 + sum2 / n2)
    fn = f / (jnp.sqrt(jnp.sum(f * f)) + 1e-8)
    sn = sim / (jnp.sqrt(jnp.sum(sim * sim, axis=-1, keepdims=True)) + 1e-8)
    scores = _dot_t(fn, sn) + gates
    iota = jax.lax.broadcasted_iota(jnp.int32, (1, 4), 1)
    v1 = jnp.max(scores)
    i1 = jnp.min(jnp.where(scores == v1, iota, 9999))
    masked = jnp.where(iota == i1, NEG, scores)
    v2 = jnp.max(masked)
    i2 = jnp.min(jnp.where(masked == v2, iota, 9999))
    e2 = jnp.exp(v2 - v1)
    w1 = 1.0 / (1.0 + e2)
    w2 = e2 / (1.0 + e2)
    l = jnp.where(iota == i1, w1, 0.0) + jnp.where(iota == i2, w2, 0.0)
    ns = jnp.sum((l > 0).astype(F32))
    return jnp.concatenate(
        [l, jnp.full((1, 1), ns, F32), jnp.zeros((1, 3), F32)], axis=1)


def _gl(g_ref, idx):
    lane = jax.lax.broadcasted_iota(jnp.int32, (1, 8), 1)
    return jnp.sum(jnp.where(lane == idx, g_ref[...], 0.0))


def _head_mask():
    d = jax.lax.broadcasted_iota(jnp.int32, (8, DIM), 1) // (DIM // 8)
    h = jax.lax.broadcasted_iota(jnp.int32, (8, DIM), 0)
    return (d == h).astype(F32)


_BIG = (2, 3, 4, 5, 7, 10, 12, 13)


def _fwd_body(*refs):
    (x_ref, wsiw_ref, wsib_ref) = refs[0:3]
    xo = refs[3:9]
    w1 = refs[9:15]
    b1 = refs[15:21]
    w2 = refs[21:27]
    b2 = refs[27:33]
    mome_in = [refs[33 + 15 * j: 33 + 15 * (j + 1)] for j in range(4)]
    (cls_ref, lng_ref, lnb_ref, wqkv_in, bqkv_ref, sawo_in, sabo_ref,
     clfw_ref, clfb_ref) = refs[93:102]
    out_ref = refs[102]
    (p_ref, o_ref, sp_ref, so_ref, k8_ref, v8_ref, bvec_ref, ctx_ref,
     q8_ref, a6_ref, g_ref, accv_ref, mv_ref, dv_ref, accd_ref, md_ref,
     dd_ref, ssum_ref, qv_ref, am_ref, ad_ref, aacc_ref, tail_ref) = \
        refs[103:126]
    wscr = refs[126:160]
    sem = refs[160]

    mome = []
    copies = []
    for j in range(4):
        mp = list(mome_in[j])
        for k, off in enumerate(_BIG):
            dst = wscr[8 * j + k]
            copies.append((mome_in[j][off], dst))
            mp[off] = dst
        mome.append(tuple(mp))
    wqkv_ref = wscr[32]
    sawo_ref = wscr[33]
    copies.append((wqkv_in, wqkv_ref))
    copies.append((sawo_in, sawo_ref))

    def _copy(i):
        src, dst = copies[i]
        return pltpu.make_async_copy(src, dst, sem.at[i])

    s = pl.program_id(0)

    @pl.when(s == 0)
    def _():
        for i in range(len(copies)):
            _copy(i).start()

    @pl.when(s == S0 - 1)
    def _():
        for i in range(0, 8):
            _copy(i).wait()

    @pl.when(s == A0 + S - 1)
    def _():
        for i in range(8, 16):
            _copy(i).wait()

    @pl.when(s == B1 + S - 1)
    def _():
        for i in range(16, 24):
            _copy(i).wait()

    @pl.when(s == A2 + S - 1)
    def _():
        for i in range(24, 34):
            _copy(i).wait()
    row8 = jax.lax.broadcasted_iota(jnp.int32, (8, 1), 0)
    mask6 = row8 < 6
    sqd = jnp.sqrt(jnp.float32(DIM))

    @pl.when(s < S0)
    def _():
        h = jnp.maximum(_dot(x_ref[...], wsiw_ref[...]) + wsib_ref[...], 0.0)
        p_ref[pl.ds(s * BMX, BMX), :] = h

        @pl.when(s == 0)
        def _():
            sp_ref[...] = jnp.zeros_like(sp_ref)
            rows = []
            for i in range(6):
                hh = _elu(_dot(xo[i][...], w1[i][...]) + b1[i][...])
                rows.append(_elu(_dot(hh, w2[i][...]) + b2[i][...]))
            o = jnp.concatenate(rows + [jnp.zeros((2, DIM), F32)], axis=0)
            o_ref[...] = o
            so_ref[...] = jnp.sum(o, axis=0, keepdims=True)

        sp_ref[...] += jnp.sum(h, axis=0, keepdims=True)

    def a_pre(mp):
        (sim, gates, wq, wk, wv, wo, n1g, s1w, s1b, n2g, s2w, s2b,
         vv, uu, wd) = mp
        o = o_ref[...]
        k8_ref[...] = _dot(o, wk[...])
        v8_ref[...] = _dot(o, wv[...])
        h2 = _elu(_dot(_rmsnorm(o, n2g[...]), s2w[...]) + s2b[...])
        h2 = jnp.where(mask6, h2, 0.0)
        bvec_ref[...] = jnp.sum(h2, axis=0, keepdims=True) / 6.0
        a = jnp.tanh(_dot(o, vv[...])) * jax.nn.sigmoid(_dot(o, uu[...]))
        sd = jnp.sum(a * wd[...], axis=1, keepdims=True)
        sd = jnp.where(mask6, sd, NEG)
        pd = jnp.exp(sd - jnp.max(sd))
        attn = pd / jnp.sum(pd)
        ctx_ref[...] = jnp.sum(attn * o, axis=0, keepdims=True)
        g_ref[...] = _gate_vec(sp_ref[...], 4096.0, so_ref[...], 6.0,
                               sim[...], gates[...])

    def a_main(mp, base):
        (sim, gates, wq, wk, wv, wo, n1g, s1w, s1b, n2g, s2w, s2b,
         vv, uu, wd) = mp
        blk = (s - base) * BM
        x = p_ref[pl.ds(blk, BM), :]
        l0, l1, l2, l3 = (_gl(g_ref, 0), _gl(g_ref, 1),
                          _gl(g_ref, 2), _gl(g_ref, 3))
        ns = _gl(g_ref, 4)
        p_ref[pl.ds(blk, BM), :] = ((l0 + l2 + l3) / ns) * x \
            + (l2 / ns) * ctx_ref[...]

        @pl.when(l0 > 0)
        def _():
            q = _dot(x, wq[...])
            sc = _dot_t(q, k8_ref[...]) / sqd
            col = jax.lax.broadcasted_iota(jnp.int32, sc.shape, 1)
            sc = jnp.where(col < 6, sc, NEG)
            e = jnp.exp(sc - jnp.max(sc, axis=1, keepdims=True))
            attn = e / jnp.sum(e, axis=1, keepdims=True)
            y = _dot(attn, v8_ref[...])
            p_ref[pl.ds(blk, BM), :] += (l0 / ns) * _dot(y, wo[...])

        @pl.when(l1 > 0)
        def _():
            a = _elu(_dot(_rmsnorm(x, n1g[...]), s1w[...]) + s1b[...])
            p_ref[pl.ds(blk, BM), :] += (l1 / ns) * (a + bvec_ref[...])

        @pl.when(s == base)
        def _():
            sp_ref[...] = jnp.zeros_like(sp_ref)

        sp_ref[...] += jnp.sum(p_ref[pl.ds(blk, BM), :], axis=0,
                               keepdims=True)

    def b_pre(mp):
        (sim, gates, wq, wk, wv, wo, n1g, s1w, s1b, n2g, s2w, s2b,
         vv, uu, wd) = mp
        o = o_ref[...]
        q8_ref[...] = _dot(o, wq[...])
        a6 = _elu(_dot(_rmsnorm(o, n1g[...]), s1w[...]) + s1b[...])
        a6_ref[...] = jnp.where(mask6, a6, 0.0)
        g_ref[...] = _gate_vec(so_ref[...], 6.0, sp_ref[...], 4096.0,
                               sim[...], gates[...])
        accv_ref[...] = jnp.zeros_like(accv_ref)
        mv_ref[...] = jnp.full_like(mv_ref, NEG)
        dv_ref[...] = jnp.zeros_like(dv_ref)
        accd_ref[...] = jnp.zeros_like(accd_ref)
        md_ref[...] = jnp.full_like(md_ref, NEG)
        dd_ref[...] = jnp.zeros_like(dd_ref)
        ssum_ref[...] = jnp.zeros_like(ssum_ref)

    def b_acc(mp, x):
        (sim, gates, wq, wk, wv, wo, n1g, s1w, s1b, n2g, s2w, s2b,
         vv, uu, wd) = mp
        l0, l1, l2 = _gl(g_ref, 0), _gl(g_ref, 1), _gl(g_ref, 2)

        @pl.when(l0 > 0)
        def _():
            k = _dot(x, wk[...])
            v = _dot(x, wv[...])
            sc = _dot_t(q8_ref[...], k) / sqd
            m_old = mv_ref[...]
            m_new = jnp.maximum(m_old, jnp.max(sc, axis=1, keepdims=True))
            alpha = jnp.exp(m_old - m_new)
            pp = jnp.exp(sc - m_new)
            mv_ref[...] = m_new
            dv_ref[...] = dv_ref[...] * alpha + jnp.sum(pp, axis=1,
                                                        keepdims=True)
            accv_ref[...] = accv_ref[...] * alpha + _dot(pp, v)

        @pl.when(l1 > 0)
        def _():
            h = _elu(_dot(_rmsnorm(x, n2g[...]), s2w[...]) + s2b[...])
            ssum_ref[...] += jnp.sum(h, axis=0, keepdims=True)

        @pl.when(l2 > 0)
        def _():
            a = jnp.tanh(_dot(x, vv[...])) * jax.nn.sigmoid(_dot(x, uu[...]))
            sc = jnp.sum(a * wd[...], axis=1, keepdims=True)
            m_old = md_ref[...]
            m_new = jnp.maximum(m_old, jnp.max(sc))
            alpha = jnp.exp(m_old - m_new)
            pp = jnp.exp(sc - m_new)
            md_ref[...] = m_new
            dd_ref[...] = dd_ref[...] * alpha + jnp.sum(pp)
            accd_ref[...] = accd_ref[...] * alpha + _dot_c0(pp, x)

    def b_comb(mp):
        (sim, gates, wq, wk, wv, wo, n1g, s1w, s1b, n2g, s2w, s2b,
         vv, uu, wd) = mp
        o = o_ref[...]
        l0, l1, l2, l3 = (_gl(g_ref, 0), _gl(g_ref, 1),
                          _gl(g_ref, 2), _gl(g_ref, 3))
        ns = _gl(g_ref, 4)
        o_ref[...] = ((l0 + l2 + l3) / ns) * o \
            + (l2 / ns) * accd_ref[...] / dd_ref[...]

        @pl.when(l0 > 0)
        def _():
            y = accv_ref[...] / dv_ref[...]
            o_ref[...] += (l0 / ns) * _dot(y, wo[...])

        @pl.when(l1 > 0)
        def _():
            o_ref[...] += (l1 / ns) * (a6_ref[...] + ssum_ref[...] / 4096.0)

        onew = jnp.where(mask6, o_ref[...], 0.0)
        o_ref[...] = onew
        so_ref[...] = jnp.sum(onew, axis=0, keepdims=True)

    hd_scale = jnp.sqrt(jnp.float32(DIM // 8))

    def attn_upd(rows, nvalid):
        m8 = _head_mask()
        y = _ln(rows, lng_ref[...], lnb_ref[...])
        k = _dot(y, wqkv_ref[:, DIM:2 * DIM]) + bqkv_ref[:, DIM:2 * DIM]
        v = _dot(y, wqkv_ref[:, 2 * DIM:]) + bqkv_ref[:, 2 * DIM:]
        sc = _dot_t(k * qv_ref[...], m8) / hd_scale
        if nvalid is not None:
            row = jax.lax.broadcasted_iota(jnp.int32, sc.shape, 0)
            sc = jnp.where(row < nvalid, sc, NEG)
        m_old = am_ref[...]
        m_new = jnp.maximum(m_old, jnp.max(sc, axis=0, keepdims=True))
        alpha = jnp.exp(m_old - m_new)
        pp = jnp.exp(sc - m_new)
        am_ref[...] = m_new
        ad_ref[...] = ad_ref[...] * alpha + jnp.sum(pp, axis=0, keepdims=True)
        pb = _dot(pp, m8)
        aacc_ref[...] = (aacc_ref[...] * _dot(alpha, m8)
                         + jnp.sum(pb * v, axis=0, keepdims=True))

    m0, m1, m2, m3 = mome

    @pl.when(s == S0 - 1)
    def _():
        a_pre(m0)

    @pl.when((s >= A0) & (s < A0 + S))
    def _():
        a_main(m0, A0)

    @pl.when(s == A0 + S - 1)
    def _():
        b_pre(m1)

    @pl.when((s >= B1) & (s < B1 + S))
    def _():
        b_acc(m1, p_ref[pl.ds((s - B1) * BM, BM), :])

    @pl.when(s == B1 + S - 1)
    def _():
        b_comb(m1)
        a_pre(m2)

    @pl.when((s >= A2) & (s < A2 + S))
    def _():
        a_main(m2, A2)

    @pl.when(s == A2 + S - 1)
    def _():
        b_pre(m3)
        ycls = _ln(cls_ref[...], lng_ref[...], lnb_ref[...])
        qv_ref[...] = _dot(ycls, wqkv_ref[:, 0:DIM]) + bqkv_ref[:, 0:DIM]
        am_ref[...] = jnp.full_like(am_ref, NEG)
        ad_ref[...] = jnp.zeros_like(ad_ref)
        aacc_ref[...] = jnp.zeros_like(aacc_ref)

    @pl.when((s >= B3) & (s < B3 + S))
    def _():
        x = p_ref[pl.ds((s - B3) * BM, BM), :]
        b_acc(m3, x)
        attn_upd(x, None)

    @pl.when(s == B3 + S - 1)
    def _():
        b_comb(m3)
        tail_ref[0:1, :] = cls_ref[...]
        tail_ref[1:7, :] = o_ref[0:6, :]
        tail_ref[7:8, :] = jnp.zeros((1, DIM), F32)
        attn_upd(tail_ref[...], 7)
        m8 = _head_mask()
        o = aacc_ref[...] / _dot(ad_ref[...], m8)
        hcls = tail_ref[0:1, :] + _dot(o, sawo_ref[...]) + sabo_ref[...]
        out_ref[...] = _dot(hcls, clfw_ref[...]) + clfb_ref[...]


def kernel(x_path, x_omic1, x_omic2, x_omic3, x_omic4, x_omic5, x_omic6,
           params):
    p = params
    xo = [x_omic1, x_omic2, x_omic3, x_omic4, x_omic5, x_omic6]

    args = [x_path, p['wsi_W'], p['wsi_b'].reshape(1, DIM)]
    args += [x.reshape(1, -1) for x in xo]
    args += [s['W1'] for s in p['sig']]
    args += [s['b1'].reshape(1, DIM) for s in p['sig']]
    args += [s['W2'] for s in p['sig']]
    args += [s['b2'].reshape(1, DIM) for s in p['sig']]
    for j in range(4):
        m = p['mome'][j]
        args += [m['gate']['sim'], m['gate']['gates'].reshape(1, 4),
                 m['coa']['Wq'], m['coa']['Wk'], m['coa']['Wv'],
                 m['coa']['Wo'],
                 m['snnf']['n1_g'].reshape(1, DIM), m['snnf']['s1_W'],
                 m['snnf']['s1_b'].reshape(1, DIM),
                 m['snnf']['n2_g'].reshape(1, DIM), m['snnf']['s2_W'],
                 m['snnf']['s2_b'].reshape(1, DIM),
                 m['damisl']['V'], m['damisl']['U'],
                 m['damisl']['w'].reshape(1, 256)]
    sa = p['sa']
    args += [sa['cls_token'].reshape(1, DIM),
             sa['ln_g'].reshape(1, DIM), sa['ln_b'].reshape(1, DIM),
             sa['Wqkv'], sa['bqkv'].reshape(1, 3 * DIM),
             sa['Wo'], sa['bo'].reshape(1, DIM),
             p['clf_W'], p['clf_b'].reshape(1, 4)]

    any_idx = set()
    for j in range(4):
        for off in (2, 3, 4, 5, 7, 10, 12, 13):
            any_idx.add(33 + 15 * j + off)
    any_idx.add(96)
    any_idx.add(98)

    in_specs = [pl.BlockSpec((BMX, 1024),
                             lambda s: (jnp.minimum(s, S0 - 1), 0))]
    for i, a in enumerate(args[1:], start=1):
        if i in any_idx:
            in_specs.append(pl.BlockSpec(memory_space=pl.ANY))
        else:
            in_specs.append(pl.BlockSpec(a.shape, lambda s: (0, 0)))

    scratch_shapes = [
        pltpu.VMEM((NP, DIM), F32),
        pltpu.VMEM((8, DIM), F32),
        pltpu.VMEM((1, DIM), F32),
        pltpu.VMEM((1, DIM), F32),
        pltpu.VMEM((8, DIM), F32),
        pltpu.VMEM((8, DIM), F32),
        pltpu.VMEM((1, DIM), F32),
        pltpu.VMEM((1, DIM), F32),
        pltpu.VMEM((8, DIM), F32),
        pltpu.VMEM((8, DIM), F32),
        pltpu.VMEM((1, 8), F32),
        pltpu.VMEM((8, DIM), F32),
        pltpu.VMEM((8, 1), F32),
        pltpu.VMEM((8, 1), F32),
        pltpu.VMEM((1, DIM), F32),
        pltpu.VMEM((1, 1), F32),
        pltpu.VMEM((1, 1), F32),
        pltpu.VMEM((1, DIM), F32),
        pltpu.VMEM((1, DIM), F32),
        pltpu.VMEM((1, 8), F32),
        pltpu.VMEM((1, 8), F32),
        pltpu.VMEM((1, DIM), F32),
        pltpu.VMEM((8, DIM), F32),
    ]
    for _ in range(4):
        scratch_shapes += [pltpu.VMEM((DIM, DIM), F32)] * 6 \
            + [pltpu.VMEM((DIM, 256), F32)] * 2
    scratch_shapes += [pltpu.VMEM((DIM, 3 * DIM), F32),
                       pltpu.VMEM((DIM, DIM), F32),
                       pltpu.SemaphoreType.DMA((34,))]

    return pl.pallas_call(
        _fwd_body,
        grid=(NSTEPS,),
        in_specs=in_specs,
        out_specs=pl.BlockSpec((1, 4), lambda s: (0, 0)),
        out_shape=jax.ShapeDtypeStruct((1, 4), F32),
        scratch_shapes=scratch_shapes,
        compiler_params=pltpu.CompilerParams(vmem_limit_bytes=100 * 2**20),
    )(*args)


# speedup vs baseline: 8.5616x; 1.0401x over previous
import jax
import jax.numpy as jnp
from jax.experimental import pallas as pl
from jax.experimental.pallas import tpu as pltpu

DIM = 512
NP = 4096
BM = 1024
BMX = 512
S = NP // BM
S0 = NP // BMX
NEG = -1e30
F32 = jnp.float32

A0 = S0
B1 = S0 + S
A2 = S0 + 2 * S
B3 = S0 + 3 * S
NSTEPS = S0 + 4 * S


def _elu(x):
    return jnp.where(x > 0, x, jnp.exp(jnp.minimum(x, 0.0)) - 1.0)


def _rmsnorm(x, g):
    return x * g / jnp.sqrt(jnp.mean(x * x, axis=-1, keepdims=True) + 1e-8)


def _dot(a, b):
    return jnp.dot(a, b, preferred_element_type=F32)


def _dot_t(a, b):
    return jax.lax.dot_general(a, b, (((1,), (1,)), ((), ())),
                               preferred_element_type=F32)


def _dot_c0(a, b):
    return jax.lax.dot_general(a, b, (((0,), (0,)), ((), ())),
                               preferred_element_type=F32)


def _ln(x, g, b):
    mu = jnp.mean(x, axis=-1, keepdims=True)
    xc = x - mu
    var = jnp.mean(xc * xc, axis=-1, keepdims=True)
    return xc / jnp.sqrt(var + 1e-5) * g + b


def _gate_vec(sum1, n1, sum2, n2, sim, gates):
    f = 0.5 * (sum1 / n1 + sum2 / n2)
    fn = f / (jnp.sqrt(jnp.sum(f * f)) + 1e-8)
    sn = sim / (jnp.sqrt(jnp.sum(sim * sim, axis=-1, keepdims=True)) + 1e-8)
    scores = _dot_t(fn, sn) + gates
    iota = jax.lax.broadcasted_iota(jnp.int32, (1, 4), 1)
    v1 = jnp.max(scores)
    i1 = jnp.min(jnp.where(scores == v1, iota, 9999))
    masked = jnp.where(iota == i1, NEG, scores)
    v2 = jnp.max(masked)
    i2 = jnp.min(jnp.where(masked == v2, iota, 9999))
    e2 = jnp.exp(v2 - v1)
    w1 = 1.0 / (1.0 + e2)
    w2 = e2 / (1.0 + e2)
    l = jnp.where(iota == i1, w1, 0.0) + jnp.where(iota == i2, w2, 0.0)
    ns = jnp.sum((l > 0).astype(F32))
    return jnp.concatenate(
        [l, jnp.full((1, 1), ns, F32), jnp.zeros((1, 3), F32)], axis=1)


def _gl(g_ref, idx):
    lane = jax.lax.broadcasted_iota(jnp.int32, (1, 8), 1)
    return jnp.sum(jnp.where(lane == idx, g_ref[...], 0.0))


def _head_mask():
    d = jax.lax.broadcasted_iota(jnp.int32, (8, DIM), 1) // (DIM // 8)
    h = jax.lax.broadcasted_iota(jnp.int32, (8, DIM), 0)
    return (d == h).astype(F32)


_BIG = (2, 3, 4, 5, 7, 10, 12, 13)


def _fwd_body(*refs):
    (x_ref, wsiw_ref, wsib_ref) = refs[0:3]
    xo = refs[3:9]
    w1 = refs[9:15]
    b1 = refs[15:21]
    w2 = refs[21:27]
    b2 = refs[27:33]
    mome_in = [refs[33 + 15 * j: 33 + 15 * (j + 1)] for j in range(4)]
    (cls_ref, lng_ref, lnb_ref, wqkv_in, bqkv_ref, sawo_in, sabo_ref,
     clfw_ref, clfb_ref) = refs[93:102]
    out_ref = refs[102]
    (p_ref, o_ref, sp_ref, so_ref, k8_ref, v8_ref, bvec_ref, ctx_ref,
     q8_ref, a6_ref, g_ref, accv_ref, mv_ref, dv_ref, accd_ref, md_ref,
     dd_ref, ssum_ref, qv_ref, am_ref, ad_ref, aacc_ref, tail_ref) = \
        refs[103:126]
    wscr = refs[126:160]
    sem = refs[160]

    mome = []
    copies = []
    for j in range(4):
        mp = list(mome_in[j])
        for k, off in enumerate(_BIG):
            dst = wscr[8 * j + k]
            copies.append((mome_in[j][off], dst))
            mp[off] = dst
        mome.append(tuple(mp))
    wqkv_ref = wscr[32]
    sawo_ref = wscr[33]
    copies.append((wqkv_in, wqkv_ref))
    copies.append((sawo_in, sawo_ref))

    def _copy(i):
        src, dst = copies[i]
        return pltpu.make_async_copy(src, dst, sem.at[i])

    s = pl.program_id(0)

    @pl.when(s == 0)
    def _():
        for i in range(len(copies)):
            _copy(i).start()

    @pl.when(s == S0 - 1)
    def _():
        for i in range(0, 8):
            _copy(i).wait()

    @pl.when(s == A0 + S - 1)
    def _():
        for i in range(8, 16):
            _copy(i).wait()

    @pl.when(s == B1 + S - 1)
    def _():
        for i in range(16, 24):
            _copy(i).wait()

    @pl.when(s == A2 + S - 1)
    def _():
        for i in range(24, 34):
            _copy(i).wait()
    row8 = jax.lax.broadcasted_iota(jnp.int32, (8, 1), 0)
    mask6 = row8 < 6
    sqd = jnp.sqrt(jnp.float32(DIM))

    @pl.when(s < S0)
    def _():
        h = jnp.maximum(_dot(x_ref[...], wsiw_ref[...]) + wsib_ref[...], 0.0)
        p_ref[pl.ds(s * BMX, BMX), :] = h

        @pl.when(s == 0)
        def _():
            sp_ref[...] = jnp.zeros_like(sp_ref)
            rows = []
            for i in range(6):
                hh = _elu(_dot(xo[i][...], w1[i][...]) + b1[i][...])
                rows.append(_elu(_dot(hh, w2[i][...]) + b2[i][...]))
            o = jnp.concatenate(rows + [jnp.zeros((2, DIM), F32)], axis=0)
            o_ref[...] = o
            so_ref[...] = jnp.sum(o, axis=0, keepdims=True)

        sp_ref[...] += jnp.sum(h, axis=0, keepdims=True)

    def a_pre(mp):
        (sim, gates, wq, wk, wv, wo, n1g, s1w, s1b, n2g, s2w, s2b,
         vv, uu, wd) = mp
        o = o_ref[...]
        k8_ref[...] = _dot(o, wk[...])
        v8_ref[...] = _dot(o, wv[...])
        h2 = _elu(_dot(_rmsnorm(o, n2g[...]), s2w[...]) + s2b[...])
        h2 = jnp.where(mask6, h2, 0.0)
        bvec_ref[...] = jnp.sum(h2, axis=0, keepdims=True) / 6.0
        a = jnp.tanh(_dot(o, vv[...])) * jax.nn.sigmoid(_dot(o, uu[...]))
        sd = jnp.sum(a * wd[...], axis=1, keepdims=True)
        sd = jnp.where(mask6, sd, NEG)
        pd = jnp.exp(sd - jnp.max(sd))
        attn = pd / jnp.sum(pd)
        ctx_ref[...] = jnp.sum(attn * o, axis=0, keepdims=True)
        g_ref[...] = _gate_vec(sp_ref[...], 4096.0, so_ref[...], 6.0,
                               sim[...], gates[...])

    def a_main(mp, base):
        (sim, gates, wq, wk, wv, wo, n1g, s1w, s1b, n2g, s2w, s2b,
         vv, uu, wd) = mp
        blk = (s - base) * BM
        x = p_ref[pl.ds(blk, BM), :]
        l0, l1, l2, l3 = (_gl(g_ref, 0), _gl(g_ref, 1),
                          _gl(g_ref, 2), _gl(g_ref, 3))
        ns = _gl(g_ref, 4)

        @pl.when(s == base)
        def _():
            sp_ref[...] = jnp.zeros_like(sp_ref)

        def base_val():
            return ((l0 + l2 + l3) / ns) * x + (l2 / ns) * ctx_ref[...]

        def coa_delta():
            q = _dot(x, wq[...])
            sc = _dot_t(q, k8_ref[...]) / sqd
            col = jax.lax.broadcasted_iota(jnp.int32, sc.shape, 1)
            sc = jnp.where(col < 6, sc, NEG)
            e = jnp.exp(sc - jnp.max(sc, axis=1, keepdims=True))
            attn = e / jnp.sum(e, axis=1, keepdims=True)
            y = _dot(attn, v8_ref[...])
            return (l0 / ns) * _dot(y, wo[...])

        def snn_val():
            a = _elu(_dot(_rmsnorm(x, n1g[...]), s1w[...]) + s1b[...])
            return (l1 / ns) * (a + bvec_ref[...])

        def emit(val):
            p_ref[pl.ds(blk, BM), :] = val
            sp_ref[...] += jnp.sum(val, axis=0, keepdims=True)

        @pl.when((l0 > 0) & (l1 > 0))
        def _():
            emit(base_val() + coa_delta() + snn_val())

        @pl.when((l0 > 0) & (l1 <= 0))
        def _():
            emit(base_val() + coa_delta())

        @pl.when((l0 <= 0) & (l1 > 0))
        def _():
            emit(base_val() + snn_val())

        @pl.when((l0 <= 0) & (l1 <= 0))
        def _():
            emit(base_val())

    def b_pre(mp):
        (sim, gates, wq, wk, wv, wo, n1g, s1w, s1b, n2g, s2w, s2b,
         vv, uu, wd) = mp
        o = o_ref[...]
        q8_ref[...] = _dot(o, wq[...])
        a6 = _elu(_dot(_rmsnorm(o, n1g[...]), s1w[...]) + s1b[...])
        a6_ref[...] = jnp.where(mask6, a6, 0.0)
        g_ref[...] = _gate_vec(so_ref[...], 6.0, sp_ref[...], 4096.0,
                               sim[...], gates[...])
        accv_ref[...] = jnp.zeros_like(accv_ref)
        mv_ref[...] = jnp.full_like(mv_ref, NEG)
        dv_ref[...] = jnp.zeros_like(dv_ref)
        accd_ref[...] = jnp.zeros_like(accd_ref)
        md_ref[...] = jnp.full_like(md_ref, NEG)
        dd_ref[...] = jnp.zeros_like(dd_ref)
        ssum_ref[...] = jnp.zeros_like(ssum_ref)

    def b_acc(mp, x):
        (sim, gates, wq, wk, wv, wo, n1g, s1w, s1b, n2g, s2w, s2b,
         vv, uu, wd) = mp
        l0, l1, l2 = _gl(g_ref, 0), _gl(g_ref, 1), _gl(g_ref, 2)

        @pl.when(l0 > 0)
        def _():
            k = _dot(x, wk[...])
            v = _dot(x, wv[...])
            sc = _dot_t(q8_ref[...], k) / sqd
            m_old = mv_ref[...]
            m_new = jnp.maximum(m_old, jnp.max(sc, axis=1, keepdims=True))
            alpha = jnp.exp(m_old - m_new)
            pp = jnp.exp(sc - m_new)
            mv_ref[...] = m_new
            dv_ref[...] = dv_ref[...] * alpha + jnp.sum(pp, axis=1,
                                                        keepdims=True)
            accv_ref[...] = accv_ref[...] * alpha + _dot(pp, v)

        @pl.when(l1 > 0)
        def _():
            h = _elu(_dot(_rmsnorm(x, n2g[...]), s2w[...]) + s2b[...])
            ssum_ref[...] += jnp.sum(h, axis=0, keepdims=True)

        @pl.when(l2 > 0)
        def _():
            a = jnp.tanh(_dot(x, vv[...])) * jax.nn.sigmoid(_dot(x, uu[...]))
            sc = jnp.sum(a * wd[...], axis=1, keepdims=True)
            m_old = md_ref[...]
            m_new = jnp.maximum(m_old, jnp.max(sc))
            alpha = jnp.exp(m_old - m_new)
            pp = jnp.exp(sc - m_new)
            md_ref[...] = m_new
            dd_ref[...] = dd_ref[...] * alpha + jnp.sum(pp)
            accd_ref[...] = accd_ref[...] * alpha + _dot_c0(pp, x)

    def b_comb(mp):
        (sim, gates, wq, wk, wv, wo, n1g, s1w, s1b, n2g, s2w, s2b,
         vv, uu, wd) = mp
        o = o_ref[...]
        l0, l1, l2, l3 = (_gl(g_ref, 0), _gl(g_ref, 1),
                          _gl(g_ref, 2), _gl(g_ref, 3))
        ns = _gl(g_ref, 4)
        o_ref[...] = ((l0 + l2 + l3) / ns) * o \
            + (l2 / ns) * accd_ref[...] / dd_ref[...]

        @pl.when(l0 > 0)
        def _():
            y = accv_ref[...] / dv_ref[...]
            o_ref[...] += (l0 / ns) * _dot(y, wo[...])

        @pl.when(l1 > 0)
        def _():
            o_ref[...] += (l1 / ns) * (a6_ref[...] + ssum_ref[...] / 4096.0)

        onew = jnp.where(mask6, o_ref[...], 0.0)
        o_ref[...] = onew
        so_ref[...] = jnp.sum(onew, axis=0, keepdims=True)

    hd_scale = jnp.sqrt(jnp.float32(DIM // 8))

    def attn_upd(rows, nvalid):
        m8 = _head_mask()
        y = _ln(rows, lng_ref[...], lnb_ref[...])
        k = _dot(y, wqkv_ref[:, DIM:2 * DIM]) + bqkv_ref[:, DIM:2 * DIM]
        v = _dot(y, wqkv_ref[:, 2 * DIM:]) + bqkv_ref[:, 2 * DIM:]
        sc = _dot_t(k * qv_ref[...], m8) / hd_scale
        if nvalid is not None:
            row = jax.lax.broadcasted_iota(jnp.int32, sc.shape, 0)
            sc = jnp.where(row < nvalid, sc, NEG)
        m_old = am_ref[...]
        m_new = jnp.maximum(m_old, jnp.max(sc, axis=0, keepdims=True))
        alpha = jnp.exp(m_old - m_new)
        pp = jnp.exp(sc - m_new)
        am_ref[...] = m_new
        ad_ref[...] = ad_ref[...] * alpha + jnp.sum(pp, axis=0, keepdims=True)
        pb = _dot(pp, m8)
        aacc_ref[...] = (aacc_ref[...] * _dot(alpha, m8)
                         + jnp.sum(pb * v, axis=0, keepdims=True))

    m0, m1, m2, m3 = mome

    @pl.when(s == S0 - 1)
    def _():
        a_pre(m0)

    @pl.when((s >= A0) & (s < A0 + S))
    def _():
        a_main(m0, A0)

    @pl.when(s == A0 + S - 1)
    def _():
        b_pre(m1)

    @pl.when((s >= B1) & (s < B1 + S))
    def _():
        b_acc(m1, p_ref[pl.ds((s - B1) * BM, BM), :])

    @pl.when(s == B1 + S - 1)
    def _():
        b_comb(m1)
        a_pre(m2)

    @pl.when((s >= A2) & (s < A2 + S))
    def _():
        a_main(m2, A2)

    @pl.when(s == A2 + S - 1)
    def _():
        b_pre(m3)
        ycls = _ln(cls_ref[...], lng_ref[...], lnb_ref[...])
        qv_ref[...] = _dot(ycls, wqkv_ref[:, 0:DIM]) + bqkv_ref[:, 0:DIM]
        am_ref[...] = jnp.full_like(am_ref, NEG)
        ad_ref[...] = jnp.zeros_like(ad_ref)
        aacc_ref[...] = jnp.zeros_like(aacc_ref)

    @pl.when((s >= B3) & (s < B3 + S))
    def _():
        x = p_ref[pl.ds((s - B3) * BM, BM), :]
        b_acc(m3, x)
        attn_upd(x, None)

    @pl.when(s == B3 + S - 1)
    def _():
        b_comb(m3)
        tail_ref[0:1, :] = cls_ref[...]
        tail_ref[1:7, :] = o_ref[0:6, :]
        tail_ref[7:8, :] = jnp.zeros((1, DIM), F32)
        attn_upd(tail_ref[...], 7)
        m8 = _head_mask()
        o = aacc_ref[...] / _dot(ad_ref[...], m8)
        hcls = tail_ref[0:1, :] + _dot(o, sawo_ref[...]) + sabo_ref[...]
        out_ref[...] = _dot(hcls, clfw_ref[...]) + clfb_ref[...]


def kernel(x_path, x_omic1, x_omic2, x_omic3, x_omic4, x_omic5, x_omic6,
           params):
    p = params
    xo = [x_omic1, x_omic2, x_omic3, x_omic4, x_omic5, x_omic6]

    args = [x_path, p['wsi_W'], p['wsi_b'].reshape(1, DIM)]
    args += [x.reshape(1, -1) for x in xo]
    args += [s['W1'] for s in p['sig']]
    args += [s['b1'].reshape(1, DIM) for s in p['sig']]
    args += [s['W2'] for s in p['sig']]
    args += [s['b2'].reshape(1, DIM) for s in p['sig']]
    for j in range(4):
        m = p['mome'][j]
        args += [m['gate']['sim'], m['gate']['gates'].reshape(1, 4),
                 m['coa']['Wq'], m['coa']['Wk'], m['coa']['Wv'],
                 m['coa']['Wo'],
                 m['snnf']['n1_g'].reshape(1, DIM), m['snnf']['s1_W'],
                 m['snnf']['s1_b'].reshape(1, DIM),
                 m['snnf']['n2_g'].reshape(1, DIM), m['snnf']['s2_W'],
                 m['snnf']['s2_b'].reshape(1, DIM),
                 m['damisl']['V'], m['damisl']['U'],
                 m['damisl']['w'].reshape(1, 256)]
    sa = p['sa']
    args += [sa['cls_token'].reshape(1, DIM),
             sa['ln_g'].reshape(1, DIM), sa['ln_b'].reshape(1, DIM),
             sa['Wqkv'], sa['bqkv'].reshape(1, 3 * DIM),
             sa['Wo'], sa['bo'].reshape(1, DIM),
             p['clf_W'], p['clf_b'].reshape(1, 4)]

    any_idx = set()
    for j in range(4):
        for off in (2, 3, 4, 5, 7, 10, 12, 13):
            any_idx.add(33 + 15 * j + off)
    any_idx.add(96)
    any_idx.add(98)

    in_specs = [pl.BlockSpec((BMX, 1024),
                             lambda s: (jnp.minimum(s, S0 - 1), 0))]
    for i, a in enumerate(args[1:], start=1):
        if i in any_idx:
            in_specs.append(pl.BlockSpec(memory_space=pl.ANY))
        else:
            in_specs.append(pl.BlockSpec(a.shape, lambda s: (0, 0)))

    scratch_shapes = [
        pltpu.VMEM((NP, DIM), F32),
        pltpu.VMEM((8, DIM), F32),
        pltpu.VMEM((1, DIM), F32),
        pltpu.VMEM((1, DIM), F32),
        pltpu.VMEM((8, DIM), F32),
        pltpu.VMEM((8, DIM), F32),
        pltpu.VMEM((1, DIM), F32),
        pltpu.VMEM((1, DIM), F32),
        pltpu.VMEM((8, DIM), F32),
        pltpu.VMEM((8, DIM), F32),
        pltpu.VMEM((1, 8), F32),
        pltpu.VMEM((8, DIM), F32),
        pltpu.VMEM((8, 1), F32),
        pltpu.VMEM((8, 1), F32),
        pltpu.VMEM((1, DIM), F32),
        pltpu.VMEM((1, 1), F32),
        pltpu.VMEM((1, 1), F32),
        pltpu.VMEM((1, DIM), F32),
        pltpu.VMEM((1, DIM), F32),
        pltpu.VMEM((1, 8), F32),
        pltpu.VMEM((1, 8), F32),
        pltpu.VMEM((1, DIM), F32),
        pltpu.VMEM((8, DIM), F32),
    ]
    for _ in range(4):
        scratch_shapes += [pltpu.VMEM((DIM, DIM), F32)] * 6 \
            + [pltpu.VMEM((DIM, 256), F32)] * 2
    scratch_shapes += [pltpu.VMEM((DIM, 3 * DIM), F32),
                       pltpu.VMEM((DIM, DIM), F32),
                       pltpu.SemaphoreType.DMA((34,))]

    return pl.pallas_call(
        _fwd_body,
        grid=(NSTEPS,),
        in_specs=in_specs,
        out_specs=pl.BlockSpec((1, 4), lambda s: (0, 0)),
        out_shape=jax.ShapeDtypeStruct((1, 4), F32),
        scratch_shapes=scratch_shapes,
        compiler_params=pltpu.CompilerParams(vmem_limit_bytes=100 * 2**20),
    )(*args)
